# baseline (speedup 1.0000x reference)
import functools

import jax
import jax.numpy as jnp
from jax import lax
from jax.experimental import pallas as pl
from jax.experimental.pallas import tpu as pltpu
from jax.experimental.pallas import tpu_sc as plsc

_N = 10000
_E = 320000
_D = 128
_G = 64
_K = 3

_NC = 2
_NS = 16
_NW = _NC * _NS
_EPW = _E // _NW
_CH = 128
_NFULL = _EPW // _CH
_TAIL = _EPW - _NFULL * _CH
_RPT = 632
_NP = _NS * _RPT


_ECH = 112
_NCHUNK = 2880
_EPAD = _NCHUNK * _ECH - _E
_CPW = _NCHUNK // _NW
_NRB = 3
_UNROLL = 3


def _make_spmm(width):
    mesh = plsc.VectorSubcoreMesh(core_axis_name="c", subcore_axis_name="s")

    scratch = [pltpu.VMEM_SHARED((_NP, width), jnp.float32)]
    scratch += [pltpu.VMEM((2, _ECH), jnp.int32) for _ in range(_NRB)]
    scratch += [pltpu.VMEM((_ECH, width), jnp.float32) for _ in range(_NRB)]
    scratch += [pltpu.SemaphoreType.DMA for _ in range(3 * _NRB)]

    @functools.partial(
        pl.kernel,
        out_type=jax.ShapeDtypeStruct((_NC * _NP, width), jnp.float32),
        mesh=mesh,
        scratch_types=scratch,
    )
    def spmm(h_hbm, pk_hbm, zeros_hbm, out_hbm, acc, *scr):
        idx = scr[0:_NRB]
        rows = scr[_NRB:2 * _NRB]
        isem = scr[2 * _NRB:3 * _NRB]
        gsem = scr[3 * _NRB:4 * _NRB]
        ssem = scr[4 * _NRB:5 * _NRB]
        c = lax.axis_index("c")
        s = lax.axis_index("s")
        wid = s * _NC + c
        base = wid * _CPW

        def istart(ch, b):
            pltpu.async_copy(pk_hbm.at[ch], idx[b], isem[b])

        def iwait(ch, b):
            pltpu.make_async_copy(pk_hbm.at[ch], idx[b], isem[b]).wait()

        def gstart(b):
            pltpu.async_copy(h_hbm.at[idx[b].at[0]], rows[b], gsem[b])

        def gwait(b):
            pltpu.make_async_copy(h_hbm.at[idx[b].at[0]], rows[b],
                                  gsem[b]).wait()

        def sstart(b):
            pltpu.async_copy(rows[b], acc.at[idx[b].at[1]], ssem[b], add=True)

        def swait(b):
            pltpu.make_async_copy(rows[b], acc.at[idx[b].at[1]],
                                  ssem[b]).wait()

        pltpu.sync_copy(zeros_hbm, acc.at[pl.ds(s * _RPT, _RPT)])
        plsc.subcore_barrier()

        istart(base + 0, 0)
        istart(base + 1, 1)
        iwait(base + 0, 0)
        gstart(0)

        def outer(i, carry):
            jb = i * _UNROLL
            for k in range(_UNROLL):
                j = jb + k
                b = k % _NRB
                b1 = (k + 1) % _NRB
                b2 = (k + 2) % _NRB

                @pl.when(j + 1 < _CPW)
                def _():
                    iwait(base + j + 1, b1)
                    gstart(b1)

                @pl.when(j >= 1)
                def _():
                    swait(b2)

                @pl.when(j + 2 < _CPW)
                def _():
                    istart(base + j + 2, b2)

                gwait(b)
                sstart(b)
            return carry

        lax.fori_loop(0, _CPW // _UNROLL, outer, 0)
        swait((_CPW - 1) % _NRB)

        plsc.subcore_barrier()
        pltpu.sync_copy(acc.at[pl.ds(s * _RPT, _RPT)],
                        out_hbm.at[pl.ds(c * _NP + s * _RPT, _RPT)])

    return spmm


def _make_deg():
    width = _D
    mesh = plsc.VectorSubcoreMesh(core_axis_name="c", subcore_axis_name="s")

    scratch = [pltpu.VMEM_SHARED((_NP, width), jnp.float32)]
    scratch += [pltpu.VMEM((2, _ECH), jnp.int32) for _ in range(_NRB)]
    scratch += [pltpu.VMEM((_ECH, width), jnp.float32)]
    scratch += [pltpu.SemaphoreType.DMA for _ in range(2 * _NRB)]

    @functools.partial(
        pl.kernel,
        out_type=jax.ShapeDtypeStruct((_NC * _NP, width), jnp.float32),
        mesh=mesh,
        scratch_types=scratch,
    )
    def degk(pk_hbm, ones_hbm, zeros_hbm, out_hbm, acc, *scr):
        idx = scr[0:_NRB]
        rows = scr[_NRB]
        isem = scr[_NRB + 1:2 * _NRB + 1]
        ssem = scr[2 * _NRB + 1:3 * _NRB + 1]
        c = lax.axis_index("c")
        s = lax.axis_index("s")
        wid = s * _NC + c
        base = wid * _CPW

        def istart(ch, b):
            pltpu.async_copy(pk_hbm.at[ch], idx[b], isem[b])

        def iwait(ch, b):
            pltpu.make_async_copy(pk_hbm.at[ch], idx[b], isem[b]).wait()

        def sstart(b):
            pltpu.async_copy(rows, acc.at[idx[b].at[1]], ssem[b], add=True)

        def swait(b):
            pltpu.make_async_copy(rows, acc.at[idx[b].at[1]],
                                  ssem[b]).wait()

        pltpu.sync_copy(zeros_hbm, acc.at[pl.ds(s * _RPT, _RPT)])
        pltpu.sync_copy(ones_hbm, rows)
        plsc.subcore_barrier()

        istart(base + 0, 0)
        istart(base + 1, 1)
        iwait(base + 0, 0)

        def outer(i, carry):
            jb = i * _UNROLL
            for k in range(_UNROLL):
                j = jb + k
                b = k % _NRB
                b1 = (k + 1) % _NRB
                b2 = (k + 2) % _NRB

                @pl.when(j + 1 < _CPW)
                def _():
                    iwait(base + j + 1, b1)

                @pl.when(j >= 1)
                def _():
                    swait(b2)

                @pl.when(j + 2 < _CPW)
                def _():
                    istart(base + j + 2, b2)

                sstart(b)
            return carry

        lax.fori_loop(0, _CPW // _UNROLL, outer, 0)
        swait((_CPW - 1) % _NRB)

        plsc.subcore_barrier()
        pltpu.sync_copy(acc.at[pl.ds(s * _RPT, _RPT)],
                        out_hbm.at[pl.ds(c * _NP + s * _RPT, _RPT)])

    return degk


_spmm = _make_spmm(_D)
_degk = _make_deg()


def _prep_body(dp_ref, x_ref, deg_ref, dis_ref, hd_ref):
    deg = dp_ref[0:_N, 0:1] + dp_ref[_NP:_NP + _N, 0:1]
    deg_ref[...] = deg
    dis = jnp.where(deg > 0, 1.0 / jnp.sqrt(jnp.maximum(deg, 1e-12)), 0.0)
    dis_ref[...] = dis
    hd_ref[...] = x_ref[...] * dis


_tc_prep = pl.pallas_call(
    _prep_body,
    out_shape=(
        jax.ShapeDtypeStruct((_N, 1), jnp.float32),
        jax.ShapeDtypeStruct((_N, 1), jnp.float32),
        jax.ShapeDtypeStruct((_N, _D), jnp.float32),
    ),
)


def _mid_body(zp_ref, dis_ref, t_ref, q_ref):
    z = zp_ref[0:_N, :] + zp_ref[_NP:_NP + _N, :]
    dis = dis_ref[...]
    t = z * dis
    t_ref[...] = t
    q_ref[...] = t * dis


_tc_mid = pl.pallas_call(
    _mid_body,
    out_shape=(
        jax.ShapeDtypeStruct((_N, _D), jnp.float32),
        jax.ShapeDtypeStruct((_N, _D), jnp.float32),
    ),
)


def _mm(a, b):
    return jnp.dot(a, b, preferred_element_type=jnp.float32)


def _bn(u, g, b):
    m = jnp.mean(u, axis=0, keepdims=True)
    v = jnp.mean((u - m) * (u - m), axis=0, keepdims=True)
    return (u - m) / jnp.sqrt(v + 1e-5) * g + b


def _stats(u):
    m = jnp.mean(u, axis=0, keepdims=True)
    v = jnp.mean((u - m) * (u - m), axis=0, keepdims=True)
    return m, v


def _tag_body(h_ref, t1_ref, t2_ref, t3_ref, tagW_ref, tagb_ref,
              u1_ref, st_ref):
    u1 = (_mm(h_ref[...], tagW_ref[0]) + _mm(t1_ref[...], tagW_ref[1])
          + _mm(t2_ref[...], tagW_ref[2]) + _mm(t3_ref[...], tagW_ref[3])
          + tagb_ref[...])
    u1_ref[...] = u1
    m1, v1 = _stats(u1)
    st_ref[...] = jnp.concatenate([m1, v1], axis=0)


_tc_tag = pl.pallas_call(
    _tag_body,
    out_shape=(
        jax.ShapeDtypeStruct((_N, _D), jnp.float32),
        jax.ShapeDtypeStruct((2, _D), jnp.float32),
    ),
)


def _lesage_body(h_ref, sp_ref, deg_ref,
                 leW1_ref, leb1_ref, leW2_ref, leW3_ref, leb3_ref,
                 sgWl_ref, sgbl_ref, sgWr_ref,
                 u2_ref, u3_ref, st_ref):
    h = h_ref[...]
    s = sp_ref[0:_N, :] + sp_ref[_NP:_NP + _N, :]
    deg = deg_ref[...]
    u2 = (deg * (_mm(h, leW1_ref[...]) + leb1_ref[...])
          - _mm(s, leW2_ref[...]) + _mm(h, leW3_ref[...]) + leb3_ref[...])
    u3 = (_mm(s / jnp.maximum(deg, 1.0), sgWl_ref[...]) + sgbl_ref[...]
          + _mm(h, sgWr_ref[...]))
    u2_ref[...] = u2
    u3_ref[...] = u3
    m2, v2 = _stats(u2)
    m3, v3 = _stats(u3)
    st_ref[...] = jnp.concatenate([m2, v2, m3, v3], axis=0)


_tc_lesage = pl.pallas_call(
    _lesage_body,
    out_shape=(
        jax.ShapeDtypeStruct((_N, _D), jnp.float32),
        jax.ShapeDtypeStruct((_N, _D), jnp.float32),
        jax.ShapeDtypeStruct((4, _D), jnp.float32),
    ),
)


def _apply_body(h_ref, u1_ref, u2_ref, u3_ref, st1_ref, st23_ref, dis_ref,
                skWci_ref, skbci_ref, skWco_ref, skbco_ref,
                bn1g_ref, bn1b_ref, bn2g_ref, bn2b_ref, bn3g_ref, bn3b_ref,
                hn_ref, hdn_ref):
    h = h_ref[...]
    dis = dis_ref[...]

    def norm(u, m, v, g, b):
        return jnp.maximum((u - m) / jnp.sqrt(v + 1e-5) * g + b, 0.0)

    o = (norm(u1_ref[...], st1_ref[0:1, :], st1_ref[1:2, :],
              bn1g_ref[...], bn1b_ref[...])
         + norm(u2_ref[...], st23_ref[0:1, :], st23_ref[1:2, :],
                bn2g_ref[...], bn2b_ref[...])
         + norm(u3_ref[...], st23_ref[2:3, :], st23_ref[3:4, :],
                bn3g_ref[...], bn3b_ref[...]))
    zl = (_mm(h, skWci_ref[...]) + skbci_ref[...]
          + _mm(o, skWco_ref[...]) + skbco_ref[...])
    z = 1.0 / (1.0 + jnp.exp(-zl))
    hn = z * o + (1.0 - z) * h
    hn_ref[...] = hn
    hdn_ref[...] = hn * dis


_tc_apply = pl.pallas_call(
    _apply_body,
    out_shape=(
        jax.ShapeDtypeStruct((_N, _D), jnp.float32),
        jax.ShapeDtypeStruct((_N, _D), jnp.float32),
    ),
)


def _readout_body(h_ref, batch_ref, eF_ref, w1h_ref, w1e_ref, b1_ref,
                  w3_ref, b3_ref, out_ref):
    h = h_ref[...]
    gids = lax.broadcasted_iota(jnp.int32, (1, _G), 1)
    onehot = (batch_ref[...] == gids).astype(jnp.float32)
    sums = lax.dot_general(onehot, h, (((0,), (0,)), ((), ())),
                           preferred_element_type=jnp.float32)
    ones_col = jnp.ones((_N, 1), jnp.float32)
    cnts = lax.dot_general(onehot, ones_col, (((0,), (0,)), ((), ())),
                           preferred_element_type=jnp.float32)
    hg = sums / jnp.maximum(cnts, 1.0)
    r = _mm(hg, w1h_ref[...]) + _mm(eF_ref[...], w1e_ref[...]) + b1_ref[...]
    r = jnp.maximum(r, 0.0)
    out_ref[...] = _mm(r, w3_ref[...]) + b3_ref[...]


_tc_readout = pl.pallas_call(
    _readout_body,
    out_shape=jax.ShapeDtypeStruct((_G, 1), jnp.float32),
)


def kernel(x, edge_index, batch, eFeature, params):
    src = edge_index[0]
    dst = edge_index[1]
    srcp = jnp.concatenate([src, jnp.zeros((_EPAD,), jnp.int32)])
    dstp = jnp.concatenate(
        [dst, _N + (jnp.arange(_EPAD, dtype=jnp.int32) % (_NP - _N))])
    pk = jnp.stack([srcp.reshape(_NCHUNK, _ECH), dstp.reshape(_NCHUNK, _ECH)],
                   axis=1)
    zeros_d = jnp.zeros((_RPT, _D), jnp.float32)
    ones_d = jnp.ones((_ECH, _D), jnp.float32)
    batch2d = batch.reshape(_N, 1)

    deg_parts = _degk(pk, ones_d, zeros_d)
    deg, dis, hd = _tc_prep(deg_parts, x)

    h = x
    p = params
    for l in (1, 2, 3):
        s_parts = _spmm(h, pk, zeros_d)
        z1p = _spmm(hd, pk, zeros_d)
        t1, q2 = _tc_mid(z1p, dis)
        z2p = _spmm(q2, pk, zeros_d)
        t2, q3 = _tc_mid(z2p, dis)
        z3p = _spmm(q3, pk, zeros_d)
        t3, _ = _tc_mid(z3p, dis)
        u1, st1 = _tc_tag(
            h, t1, t2, t3, p[f"tag{l}_W"], p[f"tag{l}_b"].reshape(1, _D),
        )
        u2, u3, st23 = _tc_lesage(
            h, s_parts, deg,
            p[f"le{l}_W1"], p[f"le{l}_b1"].reshape(1, _D),
            p[f"le{l}_W2"], p[f"le{l}_W3"], p[f"le{l}_b3"].reshape(1, _D),
            p[f"sage{l}_Wl"], p[f"sage{l}_bl"].reshape(1, _D), p[f"sage{l}_Wr"],
        )
        h, hd = _tc_apply(
            h, u1, u2, u3, st1, st23, dis,
            p[f"skip{l}_Wci"], p[f"skip{l}_bci"].reshape(1, _D),
            p[f"skip{l}_Wco"], p[f"skip{l}_bco"].reshape(1, _D),
            p[f"bn{l}1_g"].reshape(1, _D), p[f"bn{l}1_b"].reshape(1, _D),
            p[f"bn{l}2_g"].reshape(1, _D), p[f"bn{l}2_b"].reshape(1, _D),
            p[f"bn{l}3_g"].reshape(1, _D), p[f"bn{l}3_b"].reshape(1, _D),
        )

    fc1_W = params["fc1_W"]
    out = _tc_readout(
        h, batch2d, eFeature,
        fc1_W[:_D], fc1_W[_D:],
        params["fc1_b"].reshape(1, _D),
        params["fc3_W"], params["fc3_b"].reshape(1, 1),
    )
    return out


# speedup vs baseline: 1.1046x; 1.1046x over previous
import functools

import jax
import jax.numpy as jnp
from jax import lax
from jax.experimental import pallas as pl
from jax.experimental.pallas import tpu as pltpu
from jax.experimental.pallas import tpu_sc as plsc

_N = 10000
_E = 320000
_D = 128
_G = 64
_K = 3

_NC = 2
_NS = 16
_NW = _NC * _NS
_EPW = _E // _NW
_CH = 128
_NFULL = _EPW // _CH
_TAIL = _EPW - _NFULL * _CH
_RPT = 632
_NP = _NS * _RPT


_ECH = 120
_NCHUNK = 2688
_EPAD = _NCHUNK * _ECH - _E
_CPW = _NCHUNK // _NW
_NRB = 3
_UNROLL = 3


def _make_spmm(width):
    mesh = plsc.VectorSubcoreMesh(core_axis_name="c", subcore_axis_name="s")

    scratch = [pltpu.VMEM_SHARED((_NP, width), jnp.float32)]
    scratch += [pltpu.VMEM((2, _ECH), jnp.int32) for _ in range(_NRB)]
    scratch += [pltpu.VMEM((_ECH, width), jnp.float32) for _ in range(_NRB)]
    scratch += [pltpu.SemaphoreType.DMA for _ in range(3 * _NRB)]

    @functools.partial(
        pl.kernel,
        out_type=jax.ShapeDtypeStruct((_NC * _NP, width), jnp.float32),
        mesh=mesh,
        scratch_types=scratch,
    )
    def spmm(h_hbm, pk_hbm, zeros_hbm, out_hbm, acc, *scr):
        idx = scr[0:_NRB]
        rows = scr[_NRB:2 * _NRB]
        isem = scr[2 * _NRB:3 * _NRB]
        gsem = scr[3 * _NRB:4 * _NRB]
        ssem = scr[4 * _NRB:5 * _NRB]
        c = lax.axis_index("c")
        s = lax.axis_index("s")
        wid = s * _NC + c
        base = wid * _CPW

        def istart(ch, b):
            pltpu.async_copy(pk_hbm.at[ch], idx[b], isem[b])

        def iwait(ch, b):
            pltpu.make_async_copy(pk_hbm.at[ch], idx[b], isem[b]).wait()

        def gstart(b):
            pltpu.async_copy(h_hbm.at[idx[b].at[0]], rows[b], gsem[b])

        def gwait(b):
            pltpu.make_async_copy(h_hbm.at[idx[b].at[0]], rows[b],
                                  gsem[b]).wait()

        def sstart(b):
            pltpu.async_copy(rows[b], acc.at[idx[b].at[1]], ssem[b], add=True)

        def swait(b):
            pltpu.make_async_copy(rows[b], acc.at[idx[b].at[1]],
                                  ssem[b]).wait()

        pltpu.sync_copy(zeros_hbm, acc.at[pl.ds(s * _RPT, _RPT)])
        plsc.subcore_barrier()

        istart(base + 0, 0)
        istart(base + 1, 1)
        iwait(base + 0, 0)
        gstart(0)

        def outer(i, carry):
            jb = i * _UNROLL
            for k in range(_UNROLL):
                j = jb + k
                b = k % _NRB
                b1 = (k + 1) % _NRB
                b2 = (k + 2) % _NRB

                @pl.when(j + 1 < _CPW)
                def _():
                    iwait(base + j + 1, b1)
                    gstart(b1)

                @pl.when(j >= 1)
                def _():
                    swait(b2)

                @pl.when(j + 2 < _CPW)
                def _():
                    istart(base + j + 2, b2)

                gwait(b)
                sstart(b)
            return carry

        lax.fori_loop(0, _CPW // _UNROLL, outer, 0)
        swait((_CPW - 1) % _NRB)

        plsc.subcore_barrier()
        pltpu.sync_copy(acc.at[pl.ds(s * _RPT, _RPT)],
                        out_hbm.at[pl.ds(c * _NP + s * _RPT, _RPT)])

    return spmm


def _make_deg():
    width = _D
    mesh = plsc.VectorSubcoreMesh(core_axis_name="c", subcore_axis_name="s")

    scratch = [pltpu.VMEM_SHARED((_NP, width), jnp.float32)]
    scratch += [pltpu.VMEM((2, _ECH), jnp.int32) for _ in range(_NRB)]
    scratch += [pltpu.VMEM((_ECH, width), jnp.float32)]
    scratch += [pltpu.SemaphoreType.DMA for _ in range(2 * _NRB)]

    @functools.partial(
        pl.kernel,
        out_type=jax.ShapeDtypeStruct((_NC * _NP, width), jnp.float32),
        mesh=mesh,
        scratch_types=scratch,
    )
    def degk(pk_hbm, ones_hbm, zeros_hbm, out_hbm, acc, *scr):
        idx = scr[0:_NRB]
        rows = scr[_NRB]
        isem = scr[_NRB + 1:2 * _NRB + 1]
        ssem = scr[2 * _NRB + 1:3 * _NRB + 1]
        c = lax.axis_index("c")
        s = lax.axis_index("s")
        wid = s * _NC + c
        base = wid * _CPW

        def istart(ch, b):
            pltpu.async_copy(pk_hbm.at[ch], idx[b], isem[b])

        def iwait(ch, b):
            pltpu.make_async_copy(pk_hbm.at[ch], idx[b], isem[b]).wait()

        def sstart(b):
            pltpu.async_copy(rows, acc.at[idx[b].at[1]], ssem[b], add=True)

        def swait(b):
            pltpu.make_async_copy(rows, acc.at[idx[b].at[1]],
                                  ssem[b]).wait()

        pltpu.sync_copy(zeros_hbm, acc.at[pl.ds(s * _RPT, _RPT)])
        pltpu.sync_copy(ones_hbm, rows)
        plsc.subcore_barrier()

        istart(base + 0, 0)
        istart(base + 1, 1)
        iwait(base + 0, 0)

        def outer(i, carry):
            jb = i * _UNROLL
            for k in range(_UNROLL):
                j = jb + k
                b = k % _NRB
                b1 = (k + 1) % _NRB
                b2 = (k + 2) % _NRB

                @pl.when(j + 1 < _CPW)
                def _():
                    iwait(base + j + 1, b1)

                @pl.when(j >= 1)
                def _():
                    swait(b2)

                @pl.when(j + 2 < _CPW)
                def _():
                    istart(base + j + 2, b2)

                sstart(b)
            return carry

        lax.fori_loop(0, _CPW // _UNROLL, outer, 0)
        swait((_CPW - 1) % _NRB)

        plsc.subcore_barrier()
        pltpu.sync_copy(acc.at[pl.ds(s * _RPT, _RPT)],
                        out_hbm.at[pl.ds(c * _NP + s * _RPT, _RPT)])

    return degk


_spmm = _make_spmm(_D)
_degk = _make_deg()


def _prep_body(dp_ref, x_ref, deg_ref, dis_ref, hd_ref):
    deg = dp_ref[0:_N, 0:1] + dp_ref[_NP:_NP + _N, 0:1]
    deg_ref[...] = deg
    dis = jnp.where(deg > 0, 1.0 / jnp.sqrt(jnp.maximum(deg, 1e-12)), 0.0)
    dis_ref[...] = dis
    hd_ref[...] = x_ref[...] * dis


_tc_prep = pl.pallas_call(
    _prep_body,
    out_shape=(
        jax.ShapeDtypeStruct((_N, 1), jnp.float32),
        jax.ShapeDtypeStruct((_N, 1), jnp.float32),
        jax.ShapeDtypeStruct((_N, _D), jnp.float32),
    ),
)


def _mid_body(zp_ref, dis_ref, t_ref, q_ref):
    z = zp_ref[0:_N, :] + zp_ref[_NP:_NP + _N, :]
    dis = dis_ref[...]
    t = z * dis
    t_ref[...] = t
    q_ref[...] = t * dis


_tc_mid = pl.pallas_call(
    _mid_body,
    out_shape=(
        jax.ShapeDtypeStruct((_N, _D), jnp.float32),
        jax.ShapeDtypeStruct((_N, _D), jnp.float32),
    ),
)


def _mm(a, b):
    return jnp.dot(a, b, preferred_element_type=jnp.float32)


def _bn(u, g, b):
    m = jnp.mean(u, axis=0, keepdims=True)
    v = jnp.mean((u - m) * (u - m), axis=0, keepdims=True)
    return (u - m) / jnp.sqrt(v + 1e-5) * g + b


def _stats(u):
    m = jnp.mean(u, axis=0, keepdims=True)
    v = jnp.mean((u - m) * (u - m), axis=0, keepdims=True)
    return m, v


def _tag_body(h_ref, t1_ref, t2_ref, t3_ref, tagW_ref, tagb_ref,
              u1_ref, st_ref):
    u1 = (_mm(h_ref[...], tagW_ref[0]) + _mm(t1_ref[...], tagW_ref[1])
          + _mm(t2_ref[...], tagW_ref[2]) + _mm(t3_ref[...], tagW_ref[3])
          + tagb_ref[...])
    u1_ref[...] = u1
    m1, v1 = _stats(u1)
    st_ref[...] = jnp.concatenate([m1, v1], axis=0)


_tc_tag = pl.pallas_call(
    _tag_body,
    out_shape=(
        jax.ShapeDtypeStruct((_N, _D), jnp.float32),
        jax.ShapeDtypeStruct((2, _D), jnp.float32),
    ),
)


def _lesage_body(h_ref, sp_ref, deg_ref,
                 leW1_ref, leb1_ref, leW2_ref, leW3_ref, leb3_ref,
                 sgWl_ref, sgbl_ref, sgWr_ref,
                 u2_ref, u3_ref, st_ref):
    h = h_ref[...]
    s = sp_ref[0:_N, :] + sp_ref[_NP:_NP + _N, :]
    deg = deg_ref[...]
    u2 = (deg * (_mm(h, leW1_ref[...]) + leb1_ref[...])
          - _mm(s, leW2_ref[...]) + _mm(h, leW3_ref[...]) + leb3_ref[...])
    u3 = (_mm(s / jnp.maximum(deg, 1.0), sgWl_ref[...]) + sgbl_ref[...]
          + _mm(h, sgWr_ref[...]))
    u2_ref[...] = u2
    u3_ref[...] = u3
    m2, v2 = _stats(u2)
    m3, v3 = _stats(u3)
    st_ref[...] = jnp.concatenate([m2, v2, m3, v3], axis=0)


_tc_lesage = pl.pallas_call(
    _lesage_body,
    out_shape=(
        jax.ShapeDtypeStruct((_N, _D), jnp.float32),
        jax.ShapeDtypeStruct((_N, _D), jnp.float32),
        jax.ShapeDtypeStruct((4, _D), jnp.float32),
    ),
)


def _apply_body(h_ref, u1_ref, u2_ref, u3_ref, st1_ref, st23_ref, dis_ref,
                skWci_ref, skbci_ref, skWco_ref, skbco_ref,
                bn1g_ref, bn1b_ref, bn2g_ref, bn2b_ref, bn3g_ref, bn3b_ref,
                hn_ref, hdn_ref):
    h = h_ref[...]
    dis = dis_ref[...]

    def norm(u, m, v, g, b):
        return jnp.maximum((u - m) / jnp.sqrt(v + 1e-5) * g + b, 0.0)

    o = (norm(u1_ref[...], st1_ref[0:1, :], st1_ref[1:2, :],
              bn1g_ref[...], bn1b_ref[...])
         + norm(u2_ref[...], st23_ref[0:1, :], st23_ref[1:2, :],
                bn2g_ref[...], bn2b_ref[...])
         + norm(u3_ref[...], st23_ref[2:3, :], st23_ref[3:4, :],
                bn3g_ref[...], bn3b_ref[...]))
    zl = (_mm(h, skWci_ref[...]) + skbci_ref[...]
          + _mm(o, skWco_ref[...]) + skbco_ref[...])
    z = 1.0 / (1.0 + jnp.exp(-zl))
    hn = z * o + (1.0 - z) * h
    hn_ref[...] = hn
    hdn_ref[...] = hn * dis


_tc_apply = pl.pallas_call(
    _apply_body,
    out_shape=(
        jax.ShapeDtypeStruct((_N, _D), jnp.float32),
        jax.ShapeDtypeStruct((_N, _D), jnp.float32),
    ),
)


def _readout_body(h_ref, batch_ref, eF_ref, w1h_ref, w1e_ref, b1_ref,
                  w3_ref, b3_ref, out_ref):
    h = h_ref[...]
    gids = lax.broadcasted_iota(jnp.int32, (1, _G), 1)
    onehot = (batch_ref[...] == gids).astype(jnp.float32)
    sums = lax.dot_general(onehot, h, (((0,), (0,)), ((), ())),
                           preferred_element_type=jnp.float32)
    ones_col = jnp.ones((_N, 1), jnp.float32)
    cnts = lax.dot_general(onehot, ones_col, (((0,), (0,)), ((), ())),
                           preferred_element_type=jnp.float32)
    hg = sums / jnp.maximum(cnts, 1.0)
    r = _mm(hg, w1h_ref[...]) + _mm(eF_ref[...], w1e_ref[...]) + b1_ref[...]
    r = jnp.maximum(r, 0.0)
    out_ref[...] = _mm(r, w3_ref[...]) + b3_ref[...]


_tc_readout = pl.pallas_call(
    _readout_body,
    out_shape=jax.ShapeDtypeStruct((_G, 1), jnp.float32),
)


def kernel(x, edge_index, batch, eFeature, params):
    src = edge_index[0]
    dst = edge_index[1]
    srcp = jnp.concatenate([src, jnp.zeros((_EPAD,), jnp.int32)])
    dstp = jnp.concatenate(
        [dst, _N + (jnp.arange(_EPAD, dtype=jnp.int32) % (_NP - _N))])
    pk = jnp.stack([srcp.reshape(_NCHUNK, _ECH), dstp.reshape(_NCHUNK, _ECH)],
                   axis=1)
    zeros_d = jnp.zeros((_RPT, _D), jnp.float32)
    ones_d = jnp.ones((_ECH, _D), jnp.float32)
    batch2d = batch.reshape(_N, 1)

    deg_parts = _degk(pk, ones_d, zeros_d)
    deg, dis, hd = _tc_prep(deg_parts, x)

    h = x
    p = params
    for l in (1, 2, 3):
        s_parts = _spmm(h, pk, zeros_d)
        z1p = _spmm(hd, pk, zeros_d)
        t1, q2 = _tc_mid(z1p, dis)
        z2p = _spmm(q2, pk, zeros_d)
        t2, q3 = _tc_mid(z2p, dis)
        z3p = _spmm(q3, pk, zeros_d)
        t3, _ = _tc_mid(z3p, dis)
        u1, st1 = _tc_tag(
            h, t1, t2, t3, p[f"tag{l}_W"], p[f"tag{l}_b"].reshape(1, _D),
        )
        u2, u3, st23 = _tc_lesage(
            h, s_parts, deg,
            p[f"le{l}_W1"], p[f"le{l}_b1"].reshape(1, _D),
            p[f"le{l}_W2"], p[f"le{l}_W3"], p[f"le{l}_b3"].reshape(1, _D),
            p[f"sage{l}_Wl"], p[f"sage{l}_bl"].reshape(1, _D), p[f"sage{l}_Wr"],
        )
        h, hd = _tc_apply(
            h, u1, u2, u3, st1, st23, dis,
            p[f"skip{l}_Wci"], p[f"skip{l}_bci"].reshape(1, _D),
            p[f"skip{l}_Wco"], p[f"skip{l}_bco"].reshape(1, _D),
            p[f"bn{l}1_g"].reshape(1, _D), p[f"bn{l}1_b"].reshape(1, _D),
            p[f"bn{l}2_g"].reshape(1, _D), p[f"bn{l}2_b"].reshape(1, _D),
            p[f"bn{l}3_g"].reshape(1, _D), p[f"bn{l}3_b"].reshape(1, _D),
        )

    fc1_W = params["fc1_W"]
    out = _tc_readout(
        h, batch2d, eFeature,
        fc1_W[:_D], fc1_W[_D:],
        params["fc1_b"].reshape(1, _D),
        params["fc3_W"], params["fc3_b"].reshape(1, 1),
    )
    return out


# speedup vs baseline: 2.0236x; 1.8320x over previous
import functools

import jax
import jax.numpy as jnp
from jax import lax
from jax.experimental import pallas as pl
from jax.experimental.pallas import tpu as pltpu
from jax.experimental.pallas import tpu_sc as plsc

_N = 10000
_E = 320000
_D = 128
_G = 64
_K = 3

_NC = 2
_NS = 16
_NW = _NC * _NS
_EPW = _E // _NW
_CH = 128
_NFULL = _EPW // _CH
_TAIL = _EPW - _NFULL * _CH
_RPT = 632
_NP = _NS * _RPT


_ECH = 128
_NCHUNK = _E // _ECH
_CPW = _NCHUNK // _NW
_NEXTRA = _NCHUNK - _CPW * _NW
_NRB = 2
_NIB = 3
_UNROLL = 6


def _make_spmm(width):
    mesh = plsc.VectorSubcoreMesh(core_axis_name="c", subcore_axis_name="s")

    scratch = [pltpu.VMEM_SHARED((_NP, width), jnp.float32)]
    scratch += [pltpu.VMEM((2, _ECH), jnp.int32) for _ in range(_NIB)]
    scratch += [pltpu.VMEM((_ECH, width), jnp.float32) for _ in range(_NRB)]
    scratch += [pltpu.SemaphoreType.DMA for _ in range(_NIB + 2 * _NRB)]

    @functools.partial(
        pl.kernel,
        out_type=jax.ShapeDtypeStruct((_NC * _NP, width), jnp.float32),
        mesh=mesh,
        scratch_types=scratch,
    )
    def spmm(h_hbm, pk_hbm, zeros_hbm, out_hbm, acc, *scr):
        idx = scr[0:_NIB]
        rows = scr[_NIB:_NIB + _NRB]
        isem = scr[_NIB + _NRB:2 * _NIB + _NRB]
        gsem = scr[2 * _NIB + _NRB:2 * _NIB + 2 * _NRB]
        ssem = scr[2 * _NIB + 2 * _NRB:2 * _NIB + 3 * _NRB]
        c = lax.axis_index("c")
        s = lax.axis_index("s")
        wid = s * _NC + c
        base = wid * _CPW

        def istart(ch, ib):
            pltpu.async_copy(pk_hbm.at[ch], idx[ib], isem[ib])

        def iwait(ch, ib):
            pltpu.make_async_copy(pk_hbm.at[ch], idx[ib], isem[ib]).wait()

        def gstart(rb, ib):
            pltpu.async_copy(h_hbm.at[idx[ib].at[0]], rows[rb], gsem[rb])

        def gwait(rb, ib):
            pltpu.make_async_copy(h_hbm.at[idx[ib].at[0]], rows[rb],
                                  gsem[rb]).wait()

        def sstart(rb, ib):
            pltpu.async_copy(rows[rb], acc.at[idx[ib].at[1]], ssem[rb],
                             add=True)

        def swait(rb, ib):
            pltpu.make_async_copy(rows[rb], acc.at[idx[ib].at[1]],
                                  ssem[rb]).wait()

        pltpu.sync_copy(zeros_hbm, acc.at[pl.ds(s * _RPT, _RPT)])
        plsc.subcore_barrier()

        istart(base + 0, 0)
        istart(base + 1, 1)
        iwait(base + 0, 0)
        gstart(0, 0)

        def outer(i, carry):
            jb = i * _UNROLL
            for k in range(_UNROLL):
                j = jb + k
                rb = k % _NRB
                ib = k % _NIB
                rb1 = (k + 1) % _NRB
                ib1 = (k + 1) % _NIB
                ib2 = (k + 2) % _NIB

                @pl.when(j >= 1)
                def _():
                    swait(rb1, ib2)

                @pl.when(j + 2 < _CPW)
                def _():
                    istart(base + j + 2, ib2)

                @pl.when(j + 1 < _CPW)
                def _():
                    iwait(base + j + 1, ib1)
                    gstart(rb1, ib1)

                gwait(rb, ib)
                sstart(rb, ib)
            return carry

        lax.fori_loop(0, _CPW // _UNROLL, outer, 0)
        swait((_CPW - 1) % _NRB, (_CPW - 1) % _NIB)

        @pl.when(wid < _NEXTRA)
        def _():
            ch = _NW * _CPW + wid
            istart(ch, 0)
            iwait(ch, 0)
            gstart(0, 0)
            gwait(0, 0)
            sstart(0, 0)
            swait(0, 0)

        plsc.subcore_barrier()
        pltpu.sync_copy(acc.at[pl.ds(s * _RPT, _RPT)],
                        out_hbm.at[pl.ds(c * _NP + s * _RPT, _RPT)])

    return spmm


def _make_deg():
    width = _D
    mesh = plsc.VectorSubcoreMesh(core_axis_name="c", subcore_axis_name="s")

    scratch = [pltpu.VMEM_SHARED((_NP, width), jnp.float32)]
    scratch += [pltpu.VMEM((2, _ECH), jnp.int32) for _ in range(_NIB)]
    scratch += [pltpu.VMEM((_ECH, width), jnp.float32)]
    scratch += [pltpu.SemaphoreType.DMA for _ in range(2 * _NIB)]

    @functools.partial(
        pl.kernel,
        out_type=jax.ShapeDtypeStruct((_NC * _NP, width), jnp.float32),
        mesh=mesh,
        scratch_types=scratch,
    )
    def degk(pk_hbm, ones_hbm, zeros_hbm, out_hbm, acc, *scr):
        idx = scr[0:_NIB]
        rows = scr[_NIB]
        isem = scr[_NIB + 1:2 * _NIB + 1]
        ssem = scr[2 * _NIB + 1:3 * _NIB + 1]
        c = lax.axis_index("c")
        s = lax.axis_index("s")
        wid = s * _NC + c
        base = wid * _CPW

        def istart(ch, b):
            pltpu.async_copy(pk_hbm.at[ch], idx[b], isem[b])

        def iwait(ch, b):
            pltpu.make_async_copy(pk_hbm.at[ch], idx[b], isem[b]).wait()

        def sstart(b):
            pltpu.async_copy(rows, acc.at[idx[b].at[1]], ssem[b], add=True)

        def swait(b):
            pltpu.make_async_copy(rows, acc.at[idx[b].at[1]],
                                  ssem[b]).wait()

        pltpu.sync_copy(zeros_hbm, acc.at[pl.ds(s * _RPT, _RPT)])
        pltpu.sync_copy(ones_hbm, rows)
        plsc.subcore_barrier()

        istart(base + 0, 0)
        istart(base + 1, 1)
        iwait(base + 0, 0)

        def outer(i, carry):
            jb = i * _UNROLL
            for k in range(_UNROLL):
                j = jb + k
                b = k % _NIB
                b1 = (k + 1) % _NIB
                b2 = (k + 2) % _NIB

                @pl.when(j + 1 < _CPW)
                def _():
                    iwait(base + j + 1, b1)

                @pl.when(j >= 1)
                def _():
                    swait(b2)

                @pl.when(j + 2 < _CPW)
                def _():
                    istart(base + j + 2, b2)

                sstart(b)
            return carry

        lax.fori_loop(0, _CPW // _UNROLL, outer, 0)
        swait((_CPW - 1) % _NIB)

        @pl.when(wid < _NEXTRA)
        def _():
            ch = _NW * _CPW + wid
            istart(ch, 0)
            iwait(ch, 0)
            sstart(0)
            swait(0)

        plsc.subcore_barrier()
        pltpu.sync_copy(acc.at[pl.ds(s * _RPT, _RPT)],
                        out_hbm.at[pl.ds(c * _NP + s * _RPT, _RPT)])

    return degk


_spmm = _make_spmm(_D)
_degk = _make_deg()


def _prep_body(dp_ref, x_ref, deg_ref, dis_ref, hd_ref):
    deg = dp_ref[0:_N, 0:1] + dp_ref[_NP:_NP + _N, 0:1]
    deg_ref[...] = deg
    dis = jnp.where(deg > 0, 1.0 / jnp.sqrt(jnp.maximum(deg, 1e-12)), 0.0)
    dis_ref[...] = dis
    hd_ref[...] = x_ref[...] * dis


_tc_prep = pl.pallas_call(
    _prep_body,
    out_shape=(
        jax.ShapeDtypeStruct((_N, 1), jnp.float32),
        jax.ShapeDtypeStruct((_N, 1), jnp.float32),
        jax.ShapeDtypeStruct((_N, _D), jnp.float32),
    ),
)


def _mid_body(zp_ref, dis_ref, t_ref, q_ref):
    z = zp_ref[0:_N, :] + zp_ref[_NP:_NP + _N, :]
    dis = dis_ref[...]
    t = z * dis
    t_ref[...] = t
    q_ref[...] = t * dis


_tc_mid = pl.pallas_call(
    _mid_body,
    out_shape=(
        jax.ShapeDtypeStruct((_N, _D), jnp.float32),
        jax.ShapeDtypeStruct((_N, _D), jnp.float32),
    ),
)


def _mm(a, b):
    return jnp.dot(a, b, preferred_element_type=jnp.float32)


def _bn(u, g, b):
    m = jnp.mean(u, axis=0, keepdims=True)
    v = jnp.mean((u - m) * (u - m), axis=0, keepdims=True)
    return (u - m) / jnp.sqrt(v + 1e-5) * g + b


def _stats(u):
    m = jnp.mean(u, axis=0, keepdims=True)
    v = jnp.mean((u - m) * (u - m), axis=0, keepdims=True)
    return m, v


def _tag_body(h_ref, t1_ref, t2_ref, t3_ref, tagW_ref, tagb_ref,
              u1_ref, st_ref):
    u1 = (_mm(h_ref[...], tagW_ref[0]) + _mm(t1_ref[...], tagW_ref[1])
          + _mm(t2_ref[...], tagW_ref[2]) + _mm(t3_ref[...], tagW_ref[3])
          + tagb_ref[...])
    u1_ref[...] = u1
    m1, v1 = _stats(u1)
    st_ref[...] = jnp.concatenate([m1, v1], axis=0)


_tc_tag = pl.pallas_call(
    _tag_body,
    out_shape=(
        jax.ShapeDtypeStruct((_N, _D), jnp.float32),
        jax.ShapeDtypeStruct((2, _D), jnp.float32),
    ),
)


def _lesage_body(h_ref, sp_ref, deg_ref,
                 leW1_ref, leb1_ref, leW2_ref, leW3_ref, leb3_ref,
                 sgWl_ref, sgbl_ref, sgWr_ref,
                 u2_ref, u3_ref, st_ref):
    h = h_ref[...]
    s = sp_ref[0:_N, :] + sp_ref[_NP:_NP + _N, :]
    deg = deg_ref[...]
    u2 = (deg * (_mm(h, leW1_ref[...]) + leb1_ref[...])
          - _mm(s, leW2_ref[...]) + _mm(h, leW3_ref[...]) + leb3_ref[...])
    u3 = (_mm(s / jnp.maximum(deg, 1.0), sgWl_ref[...]) + sgbl_ref[...]
          + _mm(h, sgWr_ref[...]))
    u2_ref[...] = u2
    u3_ref[...] = u3
    m2, v2 = _stats(u2)
    m3, v3 = _stats(u3)
    st_ref[...] = jnp.concatenate([m2, v2, m3, v3], axis=0)


_tc_lesage = pl.pallas_call(
    _lesage_body,
    out_shape=(
        jax.ShapeDtypeStruct((_N, _D), jnp.float32),
        jax.ShapeDtypeStruct((_N, _D), jnp.float32),
        jax.ShapeDtypeStruct((4, _D), jnp.float32),
    ),
)


def _apply_body(h_ref, u1_ref, u2_ref, u3_ref, st1_ref, st23_ref, dis_ref,
                skWci_ref, skbci_ref, skWco_ref, skbco_ref,
                bn1g_ref, bn1b_ref, bn2g_ref, bn2b_ref, bn3g_ref, bn3b_ref,
                hn_ref, hdn_ref):
    h = h_ref[...]
    dis = dis_ref[...]

    def norm(u, m, v, g, b):
        return jnp.maximum((u - m) / jnp.sqrt(v + 1e-5) * g + b, 0.0)

    o = (norm(u1_ref[...], st1_ref[0:1, :], st1_ref[1:2, :],
              bn1g_ref[...], bn1b_ref[...])
         + norm(u2_ref[...], st23_ref[0:1, :], st23_ref[1:2, :],
                bn2g_ref[...], bn2b_ref[...])
         + norm(u3_ref[...], st23_ref[2:3, :], st23_ref[3:4, :],
                bn3g_ref[...], bn3b_ref[...]))
    zl = (_mm(h, skWci_ref[...]) + skbci_ref[...]
          + _mm(o, skWco_ref[...]) + skbco_ref[...])
    z = 1.0 / (1.0 + jnp.exp(-zl))
    hn = z * o + (1.0 - z) * h
    hn_ref[...] = hn
    hdn_ref[...] = hn * dis


_tc_apply = pl.pallas_call(
    _apply_body,
    out_shape=(
        jax.ShapeDtypeStruct((_N, _D), jnp.float32),
        jax.ShapeDtypeStruct((_N, _D), jnp.float32),
    ),
)


def _readout_body(h_ref, batch_ref, eF_ref, w1h_ref, w1e_ref, b1_ref,
                  w3_ref, b3_ref, out_ref):
    h = h_ref[...]
    gids = lax.broadcasted_iota(jnp.int32, (1, _G), 1)
    onehot = (batch_ref[...] == gids).astype(jnp.float32)
    sums = lax.dot_general(onehot, h, (((0,), (0,)), ((), ())),
                           preferred_element_type=jnp.float32)
    ones_col = jnp.ones((_N, 1), jnp.float32)
    cnts = lax.dot_general(onehot, ones_col, (((0,), (0,)), ((), ())),
                           preferred_element_type=jnp.float32)
    hg = sums / jnp.maximum(cnts, 1.0)
    r = _mm(hg, w1h_ref[...]) + _mm(eF_ref[...], w1e_ref[...]) + b1_ref[...]
    r = jnp.maximum(r, 0.0)
    out_ref[...] = _mm(r, w3_ref[...]) + b3_ref[...]


_tc_readout = pl.pallas_call(
    _readout_body,
    out_shape=jax.ShapeDtypeStruct((_G, 1), jnp.float32),
)


def kernel(x, edge_index, batch, eFeature, params):
    src = edge_index[0]
    dst = edge_index[1]
    pk = jnp.stack([src.reshape(_NCHUNK, _ECH), dst.reshape(_NCHUNK, _ECH)],
                   axis=1)
    zeros_d = jnp.zeros((_RPT, _D), jnp.float32)
    ones_d = jnp.ones((_ECH, _D), jnp.float32)
    batch2d = batch.reshape(_N, 1)

    deg_parts = _degk(pk, ones_d, zeros_d)
    deg, dis, hd = _tc_prep(deg_parts, x)

    h = x
    p = params
    for l in (1, 2, 3):
        s_parts = _spmm(h, pk, zeros_d)
        z1p = _spmm(hd, pk, zeros_d)
        t1, q2 = _tc_mid(z1p, dis)
        z2p = _spmm(q2, pk, zeros_d)
        t2, q3 = _tc_mid(z2p, dis)
        z3p = _spmm(q3, pk, zeros_d)
        t3, _ = _tc_mid(z3p, dis)
        u1, st1 = _tc_tag(
            h, t1, t2, t3, p[f"tag{l}_W"], p[f"tag{l}_b"].reshape(1, _D),
        )
        u2, u3, st23 = _tc_lesage(
            h, s_parts, deg,
            p[f"le{l}_W1"], p[f"le{l}_b1"].reshape(1, _D),
            p[f"le{l}_W2"], p[f"le{l}_W3"], p[f"le{l}_b3"].reshape(1, _D),
            p[f"sage{l}_Wl"], p[f"sage{l}_bl"].reshape(1, _D), p[f"sage{l}_Wr"],
        )
        h, hd = _tc_apply(
            h, u1, u2, u3, st1, st23, dis,
            p[f"skip{l}_Wci"], p[f"skip{l}_bci"].reshape(1, _D),
            p[f"skip{l}_Wco"], p[f"skip{l}_bco"].reshape(1, _D),
            p[f"bn{l}1_g"].reshape(1, _D), p[f"bn{l}1_b"].reshape(1, _D),
            p[f"bn{l}2_g"].reshape(1, _D), p[f"bn{l}2_b"].reshape(1, _D),
            p[f"bn{l}3_g"].reshape(1, _D), p[f"bn{l}3_b"].reshape(1, _D),
        )

    fc1_W = params["fc1_W"]
    out = _tc_readout(
        h, batch2d, eFeature,
        fc1_W[:_D], fc1_W[_D:],
        params["fc1_b"].reshape(1, _D),
        params["fc3_W"], params["fc3_b"].reshape(1, 1),
    )
    return out


# speedup vs baseline: 2.0587x; 1.0174x over previous
import functools

import jax
import jax.numpy as jnp
from jax import lax
from jax.experimental import pallas as pl
from jax.experimental.pallas import tpu as pltpu
from jax.experimental.pallas import tpu_sc as plsc

_N = 10000
_E = 320000
_D = 128
_G = 64
_K = 3

_NC = 2
_NS = 16
_NW = _NC * _NS
_EPW = _E // _NW
_CH = 128
_NFULL = _EPW // _CH
_TAIL = _EPW - _NFULL * _CH
_RPT = 632
_NP = _NS * _RPT


_ECH = 128
_NCHUNK = _E // _ECH
_CPW = _NCHUNK // _NW
_NEXTRA = _NCHUNK - _CPW * _NW
_NRB = 2
_NIB = 3
_UNROLL = 6


def _make_spmm(width):
    mesh = plsc.VectorSubcoreMesh(core_axis_name="c", subcore_axis_name="s")

    scratch = [pltpu.VMEM_SHARED((_NP, width), jnp.float32)]
    scratch += [pltpu.VMEM((2, _ECH), jnp.int32) for _ in range(_NIB)]
    scratch += [pltpu.VMEM((_ECH, width), jnp.float32) for _ in range(_NRB)]
    scratch += [pltpu.SemaphoreType.DMA for _ in range(_NIB + 2 * _NRB)]

    @functools.partial(
        pl.kernel,
        out_type=jax.ShapeDtypeStruct((_NC * _NP, width), jnp.float32),
        mesh=mesh,
        scratch_types=scratch,
    )
    def spmm(h_hbm, pk_hbm, zeros_hbm, out_hbm, acc, *scr):
        idx = scr[0:_NIB]
        rows = scr[_NIB:_NIB + _NRB]
        isem = scr[_NIB + _NRB:2 * _NIB + _NRB]
        gsem = scr[2 * _NIB + _NRB:2 * _NIB + 2 * _NRB]
        ssem = scr[2 * _NIB + 2 * _NRB:2 * _NIB + 3 * _NRB]
        c = lax.axis_index("c")
        s = lax.axis_index("s")
        wid = s * _NC + c
        base = wid * _CPW

        def istart(ch, ib):
            pltpu.async_copy(pk_hbm.at[ch], idx[ib], isem[ib])

        def iwait(ch, ib):
            pltpu.make_async_copy(pk_hbm.at[ch], idx[ib], isem[ib]).wait()

        def gstart(rb, ib):
            pltpu.async_copy(h_hbm.at[idx[ib].at[0]], rows[rb], gsem[rb])

        def gwait(rb, ib):
            pltpu.make_async_copy(h_hbm.at[idx[ib].at[0]], rows[rb],
                                  gsem[rb]).wait()

        def sstart(rb, ib):
            pltpu.async_copy(rows[rb], acc.at[idx[ib].at[1]], ssem[rb],
                             add=True)

        def swait(rb, ib):
            pltpu.make_async_copy(rows[rb], acc.at[idx[ib].at[1]],
                                  ssem[rb]).wait()

        pltpu.sync_copy(zeros_hbm, acc.at[pl.ds(s * _RPT, _RPT)])
        plsc.subcore_barrier()

        istart(base + 0, 0)
        istart(base + 1, 1)
        iwait(base + 0, 0)
        gstart(0, 0)

        def outer(i, carry):
            jb = i * _UNROLL
            for k in range(_UNROLL):
                j = jb + k
                rb = k % _NRB
                ib = k % _NIB
                rb1 = (k + 1) % _NRB
                ib1 = (k + 1) % _NIB
                ib2 = (k + 2) % _NIB

                @pl.when(j >= 1)
                def _():
                    swait(rb1, ib2)

                @pl.when(j + 2 < _CPW)
                def _():
                    istart(base + j + 2, ib2)

                @pl.when(j + 1 < _CPW)
                def _():
                    iwait(base + j + 1, ib1)
                    gstart(rb1, ib1)

                gwait(rb, ib)
                sstart(rb, ib)
            return carry

        lax.fori_loop(0, _CPW // _UNROLL, outer, 0)
        swait((_CPW - 1) % _NRB, (_CPW - 1) % _NIB)

        @pl.when(wid < _NEXTRA)
        def _():
            ch = _NW * _CPW + wid
            istart(ch, 0)
            iwait(ch, 0)
            gstart(0, 0)
            gwait(0, 0)
            sstart(0, 0)
            swait(0, 0)

        plsc.subcore_barrier()
        pltpu.sync_copy(acc.at[pl.ds(s * _RPT, _RPT)],
                        out_hbm.at[pl.ds(c * _NP + s * _RPT, _RPT)])

    return spmm


def _make_deg():
    width = _D
    mesh = plsc.VectorSubcoreMesh(core_axis_name="c", subcore_axis_name="s")

    scratch = [pltpu.VMEM_SHARED((_NP, width), jnp.float32)]
    scratch += [pltpu.VMEM((2, _ECH), jnp.int32) for _ in range(_NIB)]
    scratch += [pltpu.VMEM((_ECH, width), jnp.float32)]
    scratch += [pltpu.SemaphoreType.DMA for _ in range(2 * _NIB)]

    @functools.partial(
        pl.kernel,
        out_type=jax.ShapeDtypeStruct((_NC * _NP, width), jnp.float32),
        mesh=mesh,
        scratch_types=scratch,
    )
    def degk(pk_hbm, ones_hbm, zeros_hbm, out_hbm, acc, *scr):
        idx = scr[0:_NIB]
        rows = scr[_NIB]
        isem = scr[_NIB + 1:2 * _NIB + 1]
        ssem = scr[2 * _NIB + 1:3 * _NIB + 1]
        c = lax.axis_index("c")
        s = lax.axis_index("s")
        wid = s * _NC + c
        base = wid * _CPW

        def istart(ch, b):
            pltpu.async_copy(pk_hbm.at[ch], idx[b], isem[b])

        def iwait(ch, b):
            pltpu.make_async_copy(pk_hbm.at[ch], idx[b], isem[b]).wait()

        def sstart(b):
            pltpu.async_copy(rows, acc.at[idx[b].at[1]], ssem[b], add=True)

        def swait(b):
            pltpu.make_async_copy(rows, acc.at[idx[b].at[1]],
                                  ssem[b]).wait()

        pltpu.sync_copy(zeros_hbm, acc.at[pl.ds(s * _RPT, _RPT)])
        pltpu.sync_copy(ones_hbm, rows)
        plsc.subcore_barrier()

        istart(base + 0, 0)
        istart(base + 1, 1)
        iwait(base + 0, 0)

        def outer(i, carry):
            jb = i * _UNROLL
            for k in range(_UNROLL):
                j = jb + k
                b = k % _NIB
                b1 = (k + 1) % _NIB
                b2 = (k + 2) % _NIB

                @pl.when(j + 1 < _CPW)
                def _():
                    iwait(base + j + 1, b1)

                @pl.when(j >= 1)
                def _():
                    swait(b2)

                @pl.when(j + 2 < _CPW)
                def _():
                    istart(base + j + 2, b2)

                sstart(b)
            return carry

        lax.fori_loop(0, _CPW // _UNROLL, outer, 0)
        swait((_CPW - 1) % _NIB)

        @pl.when(wid < _NEXTRA)
        def _():
            ch = _NW * _CPW + wid
            istart(ch, 0)
            iwait(ch, 0)
            sstart(0)
            swait(0)

        plsc.subcore_barrier()
        pltpu.sync_copy(acc.at[pl.ds(s * _RPT, _RPT)],
                        out_hbm.at[pl.ds(c * _NP + s * _RPT, _RPT)])

    return degk


_spmm = _make_spmm(_D)
_degk = _make_deg()


def _prep_body(dp_ref, x_ref, deg_ref, dis_ref, hd_ref):
    deg = dp_ref[0:_N, 0:1] + dp_ref[_NP:_NP + _N, 0:1]
    deg_ref[...] = deg
    dis = jnp.where(deg > 0, 1.0 / jnp.sqrt(jnp.maximum(deg, 1e-12)), 0.0)
    dis_ref[...] = dis
    hd_ref[...] = x_ref[...] * dis


_tc_prep = pl.pallas_call(
    _prep_body,
    out_shape=(
        jax.ShapeDtypeStruct((_N, 1), jnp.float32),
        jax.ShapeDtypeStruct((_N, 1), jnp.float32),
        jax.ShapeDtypeStruct((_N, _D), jnp.float32),
    ),
)


def _mid_body(zp_ref, dis_ref, t_ref, q_ref):
    z = zp_ref[0:_N, :] + zp_ref[_NP:_NP + _N, :]
    dis = dis_ref[...]
    t = z * dis
    t_ref[...] = t
    q_ref[...] = t * dis


_tc_mid = pl.pallas_call(
    _mid_body,
    out_shape=(
        jax.ShapeDtypeStruct((_N, _D), jnp.float32),
        jax.ShapeDtypeStruct((_N, _D), jnp.float32),
    ),
)


def _mm(a, b):
    return jnp.dot(a, b, preferred_element_type=jnp.float32)


def _bn(u, g, b):
    m = jnp.mean(u, axis=0, keepdims=True)
    v = jnp.mean((u - m) * (u - m), axis=0, keepdims=True)
    return (u - m) / jnp.sqrt(v + 1e-5) * g + b


def _stats(u):
    m = jnp.mean(u, axis=0, keepdims=True)
    v = jnp.mean((u - m) * (u - m), axis=0, keepdims=True)
    return m, v


def _tag_body(h_ref, t1_ref, t2_ref, z3p_ref, dis_ref, tagW_ref, tagb_ref,
              u1_ref, st_ref):
    t3 = (z3p_ref[0:_N, :] + z3p_ref[_NP:_NP + _N, :]) * dis_ref[...]
    u1 = (_mm(h_ref[...], tagW_ref[0]) + _mm(t1_ref[...], tagW_ref[1])
          + _mm(t2_ref[...], tagW_ref[2]) + _mm(t3, tagW_ref[3])
          + tagb_ref[...])
    u1_ref[...] = u1
    m1, v1 = _stats(u1)
    st_ref[...] = jnp.concatenate([m1, v1], axis=0)


_tc_tag = pl.pallas_call(
    _tag_body,
    out_shape=(
        jax.ShapeDtypeStruct((_N, _D), jnp.float32),
        jax.ShapeDtypeStruct((2, _D), jnp.float32),
    ),
)


def _lesage_body(h_ref, sp_ref, deg_ref,
                 leW1_ref, leb1_ref, leW2_ref, leW3_ref, leb3_ref,
                 sgWl_ref, sgbl_ref, sgWr_ref,
                 u2_ref, u3_ref, st_ref):
    h = h_ref[...]
    s = sp_ref[0:_N, :] + sp_ref[_NP:_NP + _N, :]
    deg = deg_ref[...]
    u2 = (deg * (_mm(h, leW1_ref[...]) + leb1_ref[...])
          - _mm(s, leW2_ref[...]) + _mm(h, leW3_ref[...]) + leb3_ref[...])
    u3 = (_mm(s / jnp.maximum(deg, 1.0), sgWl_ref[...]) + sgbl_ref[...]
          + _mm(h, sgWr_ref[...]))
    u2_ref[...] = u2
    u3_ref[...] = u3
    m2, v2 = _stats(u2)
    m3, v3 = _stats(u3)
    st_ref[...] = jnp.concatenate([m2, v2, m3, v3], axis=0)


_tc_lesage = pl.pallas_call(
    _lesage_body,
    out_shape=(
        jax.ShapeDtypeStruct((_N, _D), jnp.float32),
        jax.ShapeDtypeStruct((_N, _D), jnp.float32),
        jax.ShapeDtypeStruct((4, _D), jnp.float32),
    ),
)


def _apply_body(h_ref, u1_ref, u2_ref, u3_ref, st1_ref, st23_ref, dis_ref,
                skWci_ref, skbci_ref, skWco_ref, skbco_ref,
                bn1g_ref, bn1b_ref, bn2g_ref, bn2b_ref, bn3g_ref, bn3b_ref,
                hn_ref, hdn_ref):
    h = h_ref[...]
    dis = dis_ref[...]

    def norm(u, m, v, g, b):
        return jnp.maximum((u - m) / jnp.sqrt(v + 1e-5) * g + b, 0.0)

    o = (norm(u1_ref[...], st1_ref[0:1, :], st1_ref[1:2, :],
              bn1g_ref[...], bn1b_ref[...])
         + norm(u2_ref[...], st23_ref[0:1, :], st23_ref[1:2, :],
                bn2g_ref[...], bn2b_ref[...])
         + norm(u3_ref[...], st23_ref[2:3, :], st23_ref[3:4, :],
                bn3g_ref[...], bn3b_ref[...]))
    zl = (_mm(h, skWci_ref[...]) + skbci_ref[...]
          + _mm(o, skWco_ref[...]) + skbco_ref[...])
    z = 1.0 / (1.0 + jnp.exp(-zl))
    hn = z * o + (1.0 - z) * h
    hn_ref[...] = hn
    hdn_ref[...] = hn * dis


_tc_apply = pl.pallas_call(
    _apply_body,
    out_shape=(
        jax.ShapeDtypeStruct((_N, _D), jnp.float32),
        jax.ShapeDtypeStruct((_N, _D), jnp.float32),
    ),
)


def _readout_body(h_ref, batch_ref, eF_ref, w1h_ref, w1e_ref, b1_ref,
                  w3_ref, b3_ref, out_ref):
    h = h_ref[...]
    gids = lax.broadcasted_iota(jnp.int32, (1, _G), 1)
    onehot = (batch_ref[...] == gids).astype(jnp.float32)
    sums = lax.dot_general(onehot, h, (((0,), (0,)), ((), ())),
                           preferred_element_type=jnp.float32)
    ones_col = jnp.ones((_N, 1), jnp.float32)
    cnts = lax.dot_general(onehot, ones_col, (((0,), (0,)), ((), ())),
                           preferred_element_type=jnp.float32)
    hg = sums / jnp.maximum(cnts, 1.0)
    r = _mm(hg, w1h_ref[...]) + _mm(eF_ref[...], w1e_ref[...]) + b1_ref[...]
    r = jnp.maximum(r, 0.0)
    out_ref[...] = _mm(r, w3_ref[...]) + b3_ref[...]


_tc_readout = pl.pallas_call(
    _readout_body,
    out_shape=jax.ShapeDtypeStruct((_G, 1), jnp.float32),
)


def kernel(x, edge_index, batch, eFeature, params):
    src = edge_index[0]
    dst = edge_index[1]
    pk = jnp.stack([src.reshape(_NCHUNK, _ECH), dst.reshape(_NCHUNK, _ECH)],
                   axis=1)
    zeros_d = jnp.zeros((_RPT, _D), jnp.float32)
    ones_d = jnp.ones((_ECH, _D), jnp.float32)
    batch2d = batch.reshape(_N, 1)

    deg_parts = _degk(pk, ones_d, zeros_d)
    deg, dis, hd = _tc_prep(deg_parts, x)

    h = x
    p = params
    for l in (1, 2, 3):
        s_parts = _spmm(h, pk, zeros_d)
        z1p = _spmm(hd, pk, zeros_d)
        t1, q2 = _tc_mid(z1p, dis)
        z2p = _spmm(q2, pk, zeros_d)
        t2, q3 = _tc_mid(z2p, dis)
        z3p = _spmm(q3, pk, zeros_d)
        u1, st1 = _tc_tag(
            h, t1, t2, z3p, dis, p[f"tag{l}_W"], p[f"tag{l}_b"].reshape(1, _D),
        )
        u2, u3, st23 = _tc_lesage(
            h, s_parts, deg,
            p[f"le{l}_W1"], p[f"le{l}_b1"].reshape(1, _D),
            p[f"le{l}_W2"], p[f"le{l}_W3"], p[f"le{l}_b3"].reshape(1, _D),
            p[f"sage{l}_Wl"], p[f"sage{l}_bl"].reshape(1, _D), p[f"sage{l}_Wr"],
        )
        h, hd = _tc_apply(
            h, u1, u2, u3, st1, st23, dis,
            p[f"skip{l}_Wci"], p[f"skip{l}_bci"].reshape(1, _D),
            p[f"skip{l}_Wco"], p[f"skip{l}_bco"].reshape(1, _D),
            p[f"bn{l}1_g"].reshape(1, _D), p[f"bn{l}1_b"].reshape(1, _D),
            p[f"bn{l}2_g"].reshape(1, _D), p[f"bn{l}2_b"].reshape(1, _D),
            p[f"bn{l}3_g"].reshape(1, _D), p[f"bn{l}3_b"].reshape(1, _D),
        )

    fc1_W = params["fc1_W"]
    out = _tc_readout(
        h, batch2d, eFeature,
        fc1_W[:_D], fc1_W[_D:],
        params["fc1_b"].reshape(1, _D),
        params["fc3_W"], params["fc3_b"].reshape(1, 1),
    )
    return out


# speedup vs baseline: 2.1007x; 1.0204x over previous
import functools

import jax
import jax.numpy as jnp
from jax import lax
from jax.experimental import pallas as pl
from jax.experimental.pallas import tpu as pltpu
from jax.experimental.pallas import tpu_sc as plsc

_N = 10000
_E = 320000
_D = 128
_G = 64
_K = 3

_NC = 2
_NS = 16
_NW = _NC * _NS
_EPW = _E // _NW
_CH = 128
_NFULL = _EPW // _CH
_TAIL = _EPW - _NFULL * _CH
_RPT = 632
_NP = _N


_ECH = 128
_NCHUNK = _E // _ECH
_CPW = _NCHUNK // _NW
_NEXTRA = _NCHUNK - _CPW * _NW
_NRB = 3
_NIB = 3
_UNROLL = 3


def _make_spmm(width):
    mesh = plsc.VectorSubcoreMesh(core_axis_name="c", subcore_axis_name="s")

    scratch = [pltpu.VMEM_SHARED((_NP, width), jnp.float32)]
    scratch += [pltpu.VMEM((2, _ECH), jnp.int32) for _ in range(_NIB)]
    scratch += [pltpu.VMEM((_ECH, width), jnp.float32) for _ in range(_NRB)]
    scratch += [pltpu.SemaphoreType.DMA for _ in range(_NIB + 2 * _NRB)]

    @functools.partial(
        pl.kernel,
        out_type=jax.ShapeDtypeStruct((_NC * _NP, width), jnp.float32),
        mesh=mesh,
        scratch_types=scratch,
    )
    def spmm(h_hbm, pk_hbm, zeros_hbm, out_hbm, acc, *scr):
        idx = scr[0:_NIB]
        rows = scr[_NIB:_NIB + _NRB]
        isem = scr[_NIB + _NRB:2 * _NIB + _NRB]
        gsem = scr[2 * _NIB + _NRB:2 * _NIB + 2 * _NRB]
        ssem = scr[2 * _NIB + 2 * _NRB:2 * _NIB + 3 * _NRB]
        c = lax.axis_index("c")
        s = lax.axis_index("s")
        wid = s * _NC + c
        base = wid * _CPW

        def istart(ch, ib):
            pltpu.async_copy(pk_hbm.at[ch], idx[ib], isem[ib])

        def iwait(ch, ib):
            pltpu.make_async_copy(pk_hbm.at[ch], idx[ib], isem[ib]).wait()

        def gstart(rb, ib):
            pltpu.async_copy(h_hbm.at[idx[ib].at[0]], rows[rb], gsem[rb])

        def gwait(rb, ib):
            pltpu.make_async_copy(h_hbm.at[idx[ib].at[0]], rows[rb],
                                  gsem[rb]).wait()

        def sstart(rb, ib):
            pltpu.async_copy(rows[rb], acc.at[idx[ib].at[1]], ssem[rb],
                             add=True)

        def swait(rb, ib):
            pltpu.make_async_copy(rows[rb], acc.at[idx[ib].at[1]],
                                  ssem[rb]).wait()

        @pl.when(s < 2)
        def _():
            pltpu.sync_copy(zeros_hbm, acc.at[pl.ds(s * 632, 632)])

        @pl.when(s >= 2)
        def _():
            pltpu.sync_copy(zeros_hbm.at[pl.ds(0, 624)],
                            acc.at[pl.ds(624 * s + 16, 624)])

        plsc.subcore_barrier()

        istart(base + 0, 0)
        istart(base + 1, 1)
        iwait(base + 0, 0)
        gstart(0, 0)

        def outer(i, carry):
            jb = i * _UNROLL
            for k in range(_UNROLL):
                j = jb + k
                b = k % _NRB
                b1 = (k + 1) % _NRB
                b2 = (k + 2) % _NRB

                @pl.when(j + 1 < _CPW)
                def _():
                    iwait(base + j + 1, b1)
                    gstart(b1, b1)

                @pl.when(j >= 1)
                def _():
                    swait(b2, b2)

                @pl.when(j + 2 < _CPW)
                def _():
                    istart(base + j + 2, b2)

                gwait(b, b)
                sstart(b, b)
            return carry

        lax.fori_loop(0, _CPW // _UNROLL, outer, 0)
        swait((_CPW - 1) % _NRB, (_CPW - 1) % _NIB)

        @pl.when(wid < _NEXTRA)
        def _():
            ch = _NW * _CPW + wid
            istart(ch, 0)
            iwait(ch, 0)
            gstart(0, 0)
            gwait(0, 0)
            sstart(0, 0)
            swait(0, 0)

        plsc.subcore_barrier()

        @pl.when(s < 2)
        def _():
            pltpu.sync_copy(acc.at[pl.ds(s * 632, 632)],
                            out_hbm.at[pl.ds(c * _NP + s * 632, 632)])

        @pl.when(s >= 2)
        def _():
            pltpu.sync_copy(acc.at[pl.ds(624 * s + 16, 624)],
                            out_hbm.at[pl.ds(c * _NP + 624 * s + 16, 624)])

    return spmm


def _make_deg():
    width = _D
    mesh = plsc.VectorSubcoreMesh(core_axis_name="c", subcore_axis_name="s")

    scratch = [pltpu.VMEM_SHARED((_NP, width), jnp.float32)]
    scratch += [pltpu.VMEM((2, _ECH), jnp.int32) for _ in range(_NIB)]
    scratch += [pltpu.VMEM((_ECH, width), jnp.float32)]
    scratch += [pltpu.SemaphoreType.DMA for _ in range(2 * _NIB)]

    @functools.partial(
        pl.kernel,
        out_type=jax.ShapeDtypeStruct((_NC * _NP, width), jnp.float32),
        mesh=mesh,
        scratch_types=scratch,
    )
    def degk(pk_hbm, ones_hbm, zeros_hbm, out_hbm, acc, *scr):
        idx = scr[0:_NIB]
        rows = scr[_NIB]
        isem = scr[_NIB + 1:2 * _NIB + 1]
        ssem = scr[2 * _NIB + 1:3 * _NIB + 1]
        c = lax.axis_index("c")
        s = lax.axis_index("s")
        wid = s * _NC + c
        base = wid * _CPW

        def istart(ch, b):
            pltpu.async_copy(pk_hbm.at[ch], idx[b], isem[b])

        def iwait(ch, b):
            pltpu.make_async_copy(pk_hbm.at[ch], idx[b], isem[b]).wait()

        def sstart(b):
            pltpu.async_copy(rows, acc.at[idx[b].at[1]], ssem[b], add=True)

        def swait(b):
            pltpu.make_async_copy(rows, acc.at[idx[b].at[1]],
                                  ssem[b]).wait()

        @pl.when(s < 2)
        def _():
            pltpu.sync_copy(zeros_hbm, acc.at[pl.ds(s * 632, 632)])

        @pl.when(s >= 2)
        def _():
            pltpu.sync_copy(zeros_hbm.at[pl.ds(0, 624)],
                            acc.at[pl.ds(624 * s + 16, 624)])

        pltpu.sync_copy(ones_hbm, rows)
        plsc.subcore_barrier()

        istart(base + 0, 0)
        istart(base + 1, 1)
        iwait(base + 0, 0)

        def outer(i, carry):
            jb = i * _UNROLL
            for k in range(_UNROLL):
                j = jb + k
                b = k % _NIB
                b1 = (k + 1) % _NIB
                b2 = (k + 2) % _NIB

                @pl.when(j + 1 < _CPW)
                def _():
                    iwait(base + j + 1, b1)

                @pl.when(j >= 1)
                def _():
                    swait(b2)

                @pl.when(j + 2 < _CPW)
                def _():
                    istart(base + j + 2, b2)

                sstart(b)
            return carry

        lax.fori_loop(0, _CPW // _UNROLL, outer, 0)
        swait((_CPW - 1) % _NIB)

        @pl.when(wid < _NEXTRA)
        def _():
            ch = _NW * _CPW + wid
            istart(ch, 0)
            iwait(ch, 0)
            sstart(0)
            swait(0)

        plsc.subcore_barrier()

        @pl.when(s < 2)
        def _():
            pltpu.sync_copy(acc.at[pl.ds(s * 632, 632)],
                            out_hbm.at[pl.ds(c * _NP + s * 632, 632)])

        @pl.when(s >= 2)
        def _():
            pltpu.sync_copy(acc.at[pl.ds(624 * s + 16, 624)],
                            out_hbm.at[pl.ds(c * _NP + 624 * s + 16, 624)])

    return degk


_spmm = _make_spmm(_D)
_degk = _make_deg()


def _prep_body(dp_ref, x_ref, deg_ref, dis_ref, hd_ref):
    deg = dp_ref[0:_N, 0:1] + dp_ref[_NP:_NP + _N, 0:1]
    deg_ref[...] = deg
    dis = jnp.where(deg > 0, 1.0 / jnp.sqrt(jnp.maximum(deg, 1e-12)), 0.0)
    dis_ref[...] = dis
    hd_ref[...] = x_ref[...] * dis


_tc_prep = pl.pallas_call(
    _prep_body,
    out_shape=(
        jax.ShapeDtypeStruct((_N, 1), jnp.float32),
        jax.ShapeDtypeStruct((_N, 1), jnp.float32),
        jax.ShapeDtypeStruct((_N, _D), jnp.float32),
    ),
)


def _mid_body(zp_ref, dis_ref, t_ref, q_ref):
    z = zp_ref[0:_N, :] + zp_ref[_NP:_NP + _N, :]
    dis = dis_ref[...]
    t = z * dis
    t_ref[...] = t
    q_ref[...] = t * dis


_tc_mid = pl.pallas_call(
    _mid_body,
    out_shape=(
        jax.ShapeDtypeStruct((_N, _D), jnp.float32),
        jax.ShapeDtypeStruct((_N, _D), jnp.float32),
    ),
)


def _mm(a, b):
    return jnp.dot(a, b, preferred_element_type=jnp.float32)


def _bn(u, g, b):
    m = jnp.mean(u, axis=0, keepdims=True)
    v = jnp.mean((u - m) * (u - m), axis=0, keepdims=True)
    return (u - m) / jnp.sqrt(v + 1e-5) * g + b


def _stats(u):
    m = jnp.mean(u, axis=0, keepdims=True)
    v = jnp.mean((u - m) * (u - m), axis=0, keepdims=True)
    return m, v


def _tag_body(h_ref, t1_ref, t2_ref, z3p_ref, dis_ref, tagW_ref, tagb_ref,
              u1_ref, st_ref):
    t3 = (z3p_ref[0:_N, :] + z3p_ref[_NP:_NP + _N, :]) * dis_ref[...]
    u1 = (_mm(h_ref[...], tagW_ref[0]) + _mm(t1_ref[...], tagW_ref[1])
          + _mm(t2_ref[...], tagW_ref[2]) + _mm(t3, tagW_ref[3])
          + tagb_ref[...])
    u1_ref[...] = u1
    m1, v1 = _stats(u1)
    st_ref[...] = jnp.concatenate([m1, v1], axis=0)


_tc_tag = pl.pallas_call(
    _tag_body,
    out_shape=(
        jax.ShapeDtypeStruct((_N, _D), jnp.float32),
        jax.ShapeDtypeStruct((2, _D), jnp.float32),
    ),
)


def _lesage_body(h_ref, sp_ref, deg_ref,
                 leW1_ref, leb1_ref, leW2_ref, leW3_ref, leb3_ref,
                 sgWl_ref, sgbl_ref, sgWr_ref,
                 u2_ref, u3_ref, st_ref):
    h = h_ref[...]
    s = sp_ref[0:_N, :] + sp_ref[_NP:_NP + _N, :]
    deg = deg_ref[...]
    u2 = (deg * (_mm(h, leW1_ref[...]) + leb1_ref[...])
          - _mm(s, leW2_ref[...]) + _mm(h, leW3_ref[...]) + leb3_ref[...])
    u3 = (_mm(s / jnp.maximum(deg, 1.0), sgWl_ref[...]) + sgbl_ref[...]
          + _mm(h, sgWr_ref[...]))
    u2_ref[...] = u2
    u3_ref[...] = u3
    m2, v2 = _stats(u2)
    m3, v3 = _stats(u3)
    st_ref[...] = jnp.concatenate([m2, v2, m3, v3], axis=0)


_tc_lesage = pl.pallas_call(
    _lesage_body,
    out_shape=(
        jax.ShapeDtypeStruct((_N, _D), jnp.float32),
        jax.ShapeDtypeStruct((_N, _D), jnp.float32),
        jax.ShapeDtypeStruct((4, _D), jnp.float32),
    ),
)


def _apply_body(h_ref, u1_ref, u2_ref, u3_ref, st1_ref, st23_ref, dis_ref,
                skWci_ref, skbci_ref, skWco_ref, skbco_ref,
                bn1g_ref, bn1b_ref, bn2g_ref, bn2b_ref, bn3g_ref, bn3b_ref,
                hn_ref, hdn_ref):
    h = h_ref[...]
    dis = dis_ref[...]

    def norm(u, m, v, g, b):
        return jnp.maximum((u - m) / jnp.sqrt(v + 1e-5) * g + b, 0.0)

    o = (norm(u1_ref[...], st1_ref[0:1, :], st1_ref[1:2, :],
              bn1g_ref[...], bn1b_ref[...])
         + norm(u2_ref[...], st23_ref[0:1, :], st23_ref[1:2, :],
                bn2g_ref[...], bn2b_ref[...])
         + norm(u3_ref[...], st23_ref[2:3, :], st23_ref[3:4, :],
                bn3g_ref[...], bn3b_ref[...]))
    zl = (_mm(h, skWci_ref[...]) + skbci_ref[...]
          + _mm(o, skWco_ref[...]) + skbco_ref[...])
    z = 1.0 / (1.0 + jnp.exp(-zl))
    hn = z * o + (1.0 - z) * h
    hn_ref[...] = hn
    hdn_ref[...] = hn * dis


_tc_apply = pl.pallas_call(
    _apply_body,
    out_shape=(
        jax.ShapeDtypeStruct((_N, _D), jnp.float32),
        jax.ShapeDtypeStruct((_N, _D), jnp.float32),
    ),
)


def _readout_body(h_ref, batch_ref, eF_ref, w1h_ref, w1e_ref, b1_ref,
                  w3_ref, b3_ref, out_ref):
    h = h_ref[...]
    gids = lax.broadcasted_iota(jnp.int32, (1, _G), 1)
    onehot = (batch_ref[...] == gids).astype(jnp.float32)
    sums = lax.dot_general(onehot, h, (((0,), (0,)), ((), ())),
                           preferred_element_type=jnp.float32)
    ones_col = jnp.ones((_N, 1), jnp.float32)
    cnts = lax.dot_general(onehot, ones_col, (((0,), (0,)), ((), ())),
                           preferred_element_type=jnp.float32)
    hg = sums / jnp.maximum(cnts, 1.0)
    r = _mm(hg, w1h_ref[...]) + _mm(eF_ref[...], w1e_ref[...]) + b1_ref[...]
    r = jnp.maximum(r, 0.0)
    out_ref[...] = _mm(r, w3_ref[...]) + b3_ref[...]


_tc_readout = pl.pallas_call(
    _readout_body,
    out_shape=jax.ShapeDtypeStruct((_G, 1), jnp.float32),
)


def kernel(x, edge_index, batch, eFeature, params):
    src = edge_index[0]
    dst = edge_index[1]
    pk = jnp.stack([src.reshape(_NCHUNK, _ECH), dst.reshape(_NCHUNK, _ECH)],
                   axis=1)
    zeros_d = jnp.zeros((_RPT, _D), jnp.float32)
    ones_d = jnp.ones((_ECH, _D), jnp.float32)
    batch2d = batch.reshape(_N, 1)

    deg_parts = _degk(pk, ones_d, zeros_d)
    deg, dis, hd = _tc_prep(deg_parts, x)

    h = x
    p = params
    for l in (1, 2, 3):
        s_parts = _spmm(h, pk, zeros_d)
        z1p = _spmm(hd, pk, zeros_d)
        t1, q2 = _tc_mid(z1p, dis)
        z2p = _spmm(q2, pk, zeros_d)
        t2, q3 = _tc_mid(z2p, dis)
        z3p = _spmm(q3, pk, zeros_d)
        u1, st1 = _tc_tag(
            h, t1, t2, z3p, dis, p[f"tag{l}_W"], p[f"tag{l}_b"].reshape(1, _D),
        )
        u2, u3, st23 = _tc_lesage(
            h, s_parts, deg,
            p[f"le{l}_W1"], p[f"le{l}_b1"].reshape(1, _D),
            p[f"le{l}_W2"], p[f"le{l}_W3"], p[f"le{l}_b3"].reshape(1, _D),
            p[f"sage{l}_Wl"], p[f"sage{l}_bl"].reshape(1, _D), p[f"sage{l}_Wr"],
        )
        h, hd = _tc_apply(
            h, u1, u2, u3, st1, st23, dis,
            p[f"skip{l}_Wci"], p[f"skip{l}_bci"].reshape(1, _D),
            p[f"skip{l}_Wco"], p[f"skip{l}_bco"].reshape(1, _D),
            p[f"bn{l}1_g"].reshape(1, _D), p[f"bn{l}1_b"].reshape(1, _D),
            p[f"bn{l}2_g"].reshape(1, _D), p[f"bn{l}2_b"].reshape(1, _D),
            p[f"bn{l}3_g"].reshape(1, _D), p[f"bn{l}3_b"].reshape(1, _D),
        )

    fc1_W = params["fc1_W"]
    out = _tc_readout(
        h, batch2d, eFeature,
        fc1_W[:_D], fc1_W[_D:],
        params["fc1_b"].reshape(1, _D),
        params["fc3_W"], params["fc3_b"].reshape(1, 1),
    )
    return out


# speedup vs baseline: 2.1186x; 1.0085x over previous
import functools

import jax
import jax.numpy as jnp
from jax import lax
from jax.experimental import pallas as pl
from jax.experimental.pallas import tpu as pltpu
from jax.experimental.pallas import tpu_sc as plsc

_N = 10000
_E = 320000
_D = 128
_G = 64
_K = 3

_NC = 2
_NS = 16
_NW = _NC * _NS
_EPW = _E // _NW
_CH = 128
_NFULL = _EPW // _CH
_TAIL = _EPW - _NFULL * _CH
_RPT = 632
_NP = _N


_ECH = 128
_NCHUNK = _E // _ECH
_CPW = _NCHUNK // _NW
_NEXTRA = _NCHUNK - _CPW * _NW
_NRB = 3
_NIB = 3
_UNROLL = 3


def _make_spmm(width):
    mesh = plsc.VectorSubcoreMesh(core_axis_name="c", subcore_axis_name="s")

    scratch = [pltpu.VMEM_SHARED((_NP, width), jnp.float32)]
    scratch += [pltpu.VMEM((2, _ECH), jnp.int32) for _ in range(_NIB)]
    scratch += [pltpu.VMEM((_ECH, width), jnp.float32) for _ in range(_NRB)]
    scratch += [pltpu.SemaphoreType.DMA for _ in range(_NIB + 2 * _NRB)]

    @functools.partial(
        pl.kernel,
        out_type=jax.ShapeDtypeStruct((_NC * _NP, width), jnp.float32),
        mesh=mesh,
        scratch_types=scratch,
    )
    def spmm(h_hbm, pk_hbm, zeros_hbm, out_hbm, acc, *scr):
        idx = scr[0:_NIB]
        rows = scr[_NIB:_NIB + _NRB]
        isem = scr[_NIB + _NRB:2 * _NIB + _NRB]
        gsem = scr[2 * _NIB + _NRB:2 * _NIB + 2 * _NRB]
        ssem = scr[2 * _NIB + 2 * _NRB:2 * _NIB + 3 * _NRB]
        c = lax.axis_index("c")
        s = lax.axis_index("s")
        wid = s * _NC + c
        base = wid * _CPW

        def istart(ch, ib):
            pltpu.async_copy(pk_hbm.at[ch], idx[ib], isem[ib])

        def iwait(ch, ib):
            pltpu.make_async_copy(pk_hbm.at[ch], idx[ib], isem[ib]).wait()

        def gstart(rb, ib):
            pltpu.async_copy(h_hbm.at[idx[ib].at[0]], rows[rb], gsem[rb])

        def gwait(rb, ib):
            pltpu.make_async_copy(h_hbm.at[idx[ib].at[0]], rows[rb],
                                  gsem[rb]).wait()

        def sstart(rb, ib):
            pltpu.async_copy(rows[rb], acc.at[idx[ib].at[1]], ssem[rb],
                             add=True)

        def swait(rb, ib):
            pltpu.make_async_copy(rows[rb], acc.at[idx[ib].at[1]],
                                  ssem[rb]).wait()

        @pl.when(s < 2)
        def _():
            pltpu.sync_copy(zeros_hbm, acc.at[pl.ds(s * 632, 632)])

        @pl.when(s >= 2)
        def _():
            pltpu.sync_copy(zeros_hbm.at[pl.ds(0, 624)],
                            acc.at[pl.ds(624 * s + 16, 624)])

        istart(base + 0, 0)
        istart(base + 1, 1)
        iwait(base + 0, 0)
        gstart(0, 0)
        plsc.subcore_barrier()

        def outer(i, carry):
            jb = i * _UNROLL
            for k in range(_UNROLL):
                j = jb + k
                b = k % _NRB
                b1 = (k + 1) % _NRB
                b2 = (k + 2) % _NRB

                @pl.when(j + 1 < _CPW)
                def _():
                    iwait(base + j + 1, b1)
                    gstart(b1, b1)

                @pl.when(j >= 1)
                def _():
                    swait(b2, b2)

                @pl.when(j + 2 < _CPW)
                def _():
                    istart(base + j + 2, b2)

                gwait(b, b)
                sstart(b, b)
            return carry

        lax.fori_loop(0, _CPW // _UNROLL, outer, 0)
        swait((_CPW - 1) % _NRB, (_CPW - 1) % _NIB)

        @pl.when(wid < _NEXTRA)
        def _():
            ch = _NW * _CPW + wid
            istart(ch, 0)
            iwait(ch, 0)
            gstart(0, 0)
            gwait(0, 0)
            sstart(0, 0)
            swait(0, 0)

        plsc.subcore_barrier()

        @pl.when(s < 2)
        def _():
            pltpu.sync_copy(acc.at[pl.ds(s * 632, 632)],
                            out_hbm.at[pl.ds(c * _NP + s * 632, 632)])

        @pl.when(s >= 2)
        def _():
            pltpu.sync_copy(acc.at[pl.ds(624 * s + 16, 624)],
                            out_hbm.at[pl.ds(c * _NP + 624 * s + 16, 624)])

    return spmm


def _make_deg():
    width = _D
    mesh = plsc.VectorSubcoreMesh(core_axis_name="c", subcore_axis_name="s")

    scratch = [pltpu.VMEM_SHARED((_NP, width), jnp.float32)]
    scratch += [pltpu.VMEM((2, _ECH), jnp.int32) for _ in range(_NIB)]
    scratch += [pltpu.VMEM((_ECH, width), jnp.float32)]
    scratch += [pltpu.SemaphoreType.DMA for _ in range(2 * _NIB)]

    @functools.partial(
        pl.kernel,
        out_type=jax.ShapeDtypeStruct((_NC * _NP, width), jnp.float32),
        mesh=mesh,
        scratch_types=scratch,
    )
    def degk(pk_hbm, ones_hbm, zeros_hbm, out_hbm, acc, *scr):
        idx = scr[0:_NIB]
        rows = scr[_NIB]
        isem = scr[_NIB + 1:2 * _NIB + 1]
        ssem = scr[2 * _NIB + 1:3 * _NIB + 1]
        c = lax.axis_index("c")
        s = lax.axis_index("s")
        wid = s * _NC + c
        base = wid * _CPW

        def istart(ch, b):
            pltpu.async_copy(pk_hbm.at[ch], idx[b], isem[b])

        def iwait(ch, b):
            pltpu.make_async_copy(pk_hbm.at[ch], idx[b], isem[b]).wait()

        def sstart(b):
            pltpu.async_copy(rows, acc.at[idx[b].at[1]], ssem[b], add=True)

        def swait(b):
            pltpu.make_async_copy(rows, acc.at[idx[b].at[1]],
                                  ssem[b]).wait()

        @pl.when(s < 2)
        def _():
            pltpu.sync_copy(zeros_hbm, acc.at[pl.ds(s * 632, 632)])

        @pl.when(s >= 2)
        def _():
            pltpu.sync_copy(zeros_hbm.at[pl.ds(0, 624)],
                            acc.at[pl.ds(624 * s + 16, 624)])

        pltpu.sync_copy(ones_hbm, rows)
        plsc.subcore_barrier()

        istart(base + 0, 0)
        istart(base + 1, 1)
        iwait(base + 0, 0)

        def outer(i, carry):
            jb = i * _UNROLL
            for k in range(_UNROLL):
                j = jb + k
                b = k % _NIB
                b1 = (k + 1) % _NIB
                b2 = (k + 2) % _NIB

                @pl.when(j + 1 < _CPW)
                def _():
                    iwait(base + j + 1, b1)

                @pl.when(j >= 1)
                def _():
                    swait(b2)

                @pl.when(j + 2 < _CPW)
                def _():
                    istart(base + j + 2, b2)

                sstart(b)
            return carry

        lax.fori_loop(0, _CPW // _UNROLL, outer, 0)
        swait((_CPW - 1) % _NIB)

        @pl.when(wid < _NEXTRA)
        def _():
            ch = _NW * _CPW + wid
            istart(ch, 0)
            iwait(ch, 0)
            sstart(0)
            swait(0)

        plsc.subcore_barrier()

        @pl.when(s < 2)
        def _():
            pltpu.sync_copy(acc.at[pl.ds(s * 632, 632)],
                            out_hbm.at[pl.ds(c * _NP + s * 632, 632)])

        @pl.when(s >= 2)
        def _():
            pltpu.sync_copy(acc.at[pl.ds(624 * s + 16, 624)],
                            out_hbm.at[pl.ds(c * _NP + 624 * s + 16, 624)])

    return degk


_spmm = _make_spmm(_D)
_degk = _make_deg()


def _prep_body(dp_ref, x_ref, deg_ref, dis_ref, hd_ref):
    deg = dp_ref[0:_N, 0:1] + dp_ref[_NP:_NP + _N, 0:1]
    deg_ref[...] = deg
    dis = jnp.where(deg > 0, 1.0 / jnp.sqrt(jnp.maximum(deg, 1e-12)), 0.0)
    dis_ref[...] = dis
    hd_ref[...] = x_ref[...] * dis


_tc_prep = pl.pallas_call(
    _prep_body,
    out_shape=(
        jax.ShapeDtypeStruct((_N, 1), jnp.float32),
        jax.ShapeDtypeStruct((_N, 1), jnp.float32),
        jax.ShapeDtypeStruct((_N, _D), jnp.float32),
    ),
)


def _mid_body(zp_ref, dis_ref, t_ref, q_ref):
    z = zp_ref[0:_N, :] + zp_ref[_NP:_NP + _N, :]
    dis = dis_ref[...]
    t = z * dis
    t_ref[...] = t
    q_ref[...] = t * dis


_tc_mid = pl.pallas_call(
    _mid_body,
    out_shape=(
        jax.ShapeDtypeStruct((_N, _D), jnp.float32),
        jax.ShapeDtypeStruct((_N, _D), jnp.float32),
    ),
)


def _mm(a, b):
    return jnp.dot(a, b, preferred_element_type=jnp.float32)


def _bn(u, g, b):
    m = jnp.mean(u, axis=0, keepdims=True)
    v = jnp.mean((u - m) * (u - m), axis=0, keepdims=True)
    return (u - m) / jnp.sqrt(v + 1e-5) * g + b


def _stats(u):
    m = jnp.mean(u, axis=0, keepdims=True)
    v = jnp.mean((u - m) * (u - m), axis=0, keepdims=True)
    return m, v


def _tag_body(h_ref, t1_ref, t2_ref, z3p_ref, dis_ref, tagW_ref, tagb_ref,
              u1_ref, st_ref):
    t3 = (z3p_ref[0:_N, :] + z3p_ref[_NP:_NP + _N, :]) * dis_ref[...]
    u1 = (_mm(h_ref[...], tagW_ref[0]) + _mm(t1_ref[...], tagW_ref[1])
          + _mm(t2_ref[...], tagW_ref[2]) + _mm(t3, tagW_ref[3])
          + tagb_ref[...])
    u1_ref[...] = u1
    m1, v1 = _stats(u1)
    st_ref[...] = jnp.concatenate([m1, v1], axis=0)


_tc_tag = pl.pallas_call(
    _tag_body,
    out_shape=(
        jax.ShapeDtypeStruct((_N, _D), jnp.float32),
        jax.ShapeDtypeStruct((2, _D), jnp.float32),
    ),
)


def _lesage_body(h_ref, sp_ref, deg_ref,
                 leW1_ref, leb1_ref, leW2_ref, leW3_ref, leb3_ref,
                 sgWl_ref, sgbl_ref, sgWr_ref,
                 u2_ref, u3_ref, st_ref):
    h = h_ref[...]
    s = sp_ref[0:_N, :] + sp_ref[_NP:_NP + _N, :]
    deg = deg_ref[...]
    u2 = (deg * (_mm(h, leW1_ref[...]) + leb1_ref[...])
          - _mm(s, leW2_ref[...]) + _mm(h, leW3_ref[...]) + leb3_ref[...])
    u3 = (_mm(s / jnp.maximum(deg, 1.0), sgWl_ref[...]) + sgbl_ref[...]
          + _mm(h, sgWr_ref[...]))
    u2_ref[...] = u2
    u3_ref[...] = u3
    m2, v2 = _stats(u2)
    m3, v3 = _stats(u3)
    st_ref[...] = jnp.concatenate([m2, v2, m3, v3], axis=0)


_tc_lesage = pl.pallas_call(
    _lesage_body,
    out_shape=(
        jax.ShapeDtypeStruct((_N, _D), jnp.float32),
        jax.ShapeDtypeStruct((_N, _D), jnp.float32),
        jax.ShapeDtypeStruct((4, _D), jnp.float32),
    ),
)


def _apply_body(h_ref, u1_ref, u2_ref, u3_ref, st1_ref, st23_ref, dis_ref,
                skWci_ref, skbci_ref, skWco_ref, skbco_ref,
                bn1g_ref, bn1b_ref, bn2g_ref, bn2b_ref, bn3g_ref, bn3b_ref,
                hn_ref, hdn_ref):
    h = h_ref[...]
    dis = dis_ref[...]

    def norm(u, m, v, g, b):
        return jnp.maximum((u - m) / jnp.sqrt(v + 1e-5) * g + b, 0.0)

    o = (norm(u1_ref[...], st1_ref[0:1, :], st1_ref[1:2, :],
              bn1g_ref[...], bn1b_ref[...])
         + norm(u2_ref[...], st23_ref[0:1, :], st23_ref[1:2, :],
                bn2g_ref[...], bn2b_ref[...])
         + norm(u3_ref[...], st23_ref[2:3, :], st23_ref[3:4, :],
                bn3g_ref[...], bn3b_ref[...]))
    zl = (_mm(h, skWci_ref[...]) + skbci_ref[...]
          + _mm(o, skWco_ref[...]) + skbco_ref[...])
    z = 1.0 / (1.0 + jnp.exp(-zl))
    hn = z * o + (1.0 - z) * h
    hn_ref[...] = hn
    hdn_ref[...] = hn * dis


_tc_apply = pl.pallas_call(
    _apply_body,
    out_shape=(
        jax.ShapeDtypeStruct((_N, _D), jnp.float32),
        jax.ShapeDtypeStruct((_N, _D), jnp.float32),
    ),
)


def _apply_out_body(h_ref, u1_ref, u2_ref, u3_ref, st1_ref, st23_ref,
                    skWci_ref, skbci_ref, skWco_ref, skbco_ref,
                    bn1g_ref, bn1b_ref, bn2g_ref, bn2b_ref,
                    bn3g_ref, bn3b_ref,
                    batch_ref, eF_ref, w1h_ref, w1e_ref, b1_ref,
                    w3_ref, b3_ref, out_ref):
    h = h_ref[...]

    def norm(u, m, v, g, b):
        return jnp.maximum((u - m) / jnp.sqrt(v + 1e-5) * g + b, 0.0)

    o = (norm(u1_ref[...], st1_ref[0:1, :], st1_ref[1:2, :],
              bn1g_ref[...], bn1b_ref[...])
         + norm(u2_ref[...], st23_ref[0:1, :], st23_ref[1:2, :],
                bn2g_ref[...], bn2b_ref[...])
         + norm(u3_ref[...], st23_ref[2:3, :], st23_ref[3:4, :],
                bn3g_ref[...], bn3b_ref[...]))
    zl = (_mm(h, skWci_ref[...]) + skbci_ref[...]
          + _mm(o, skWco_ref[...]) + skbco_ref[...])
    z = 1.0 / (1.0 + jnp.exp(-zl))
    hn = z * o + (1.0 - z) * h

    gids = lax.broadcasted_iota(jnp.int32, (1, _G), 1)
    onehot = (batch_ref[...] == gids).astype(jnp.float32)
    sums = lax.dot_general(onehot, hn, (((0,), (0,)), ((), ())),
                           preferred_element_type=jnp.float32)
    ones_col = jnp.ones((_N, 1), jnp.float32)
    cnts = lax.dot_general(onehot, ones_col, (((0,), (0,)), ((), ())),
                           preferred_element_type=jnp.float32)
    hg = sums / jnp.maximum(cnts, 1.0)
    r = _mm(hg, w1h_ref[...]) + _mm(eF_ref[...], w1e_ref[...]) + b1_ref[...]
    r = jnp.maximum(r, 0.0)
    out_ref[...] = _mm(r, w3_ref[...]) + b3_ref[...]


_tc_apply_out = pl.pallas_call(
    _apply_out_body,
    out_shape=jax.ShapeDtypeStruct((_G, 1), jnp.float32),
)


def _readout_body(h_ref, batch_ref, eF_ref, w1h_ref, w1e_ref, b1_ref,
                  w3_ref, b3_ref, out_ref):
    h = h_ref[...]
    gids = lax.broadcasted_iota(jnp.int32, (1, _G), 1)
    onehot = (batch_ref[...] == gids).astype(jnp.float32)
    sums = lax.dot_general(onehot, h, (((0,), (0,)), ((), ())),
                           preferred_element_type=jnp.float32)
    ones_col = jnp.ones((_N, 1), jnp.float32)
    cnts = lax.dot_general(onehot, ones_col, (((0,), (0,)), ((), ())),
                           preferred_element_type=jnp.float32)
    hg = sums / jnp.maximum(cnts, 1.0)
    r = _mm(hg, w1h_ref[...]) + _mm(eF_ref[...], w1e_ref[...]) + b1_ref[...]
    r = jnp.maximum(r, 0.0)
    out_ref[...] = _mm(r, w3_ref[...]) + b3_ref[...]


_tc_readout = pl.pallas_call(
    _readout_body,
    out_shape=jax.ShapeDtypeStruct((_G, 1), jnp.float32),
)


def kernel(x, edge_index, batch, eFeature, params):
    src = edge_index[0]
    dst = edge_index[1]
    pk = jnp.stack([src.reshape(_NCHUNK, _ECH), dst.reshape(_NCHUNK, _ECH)],
                   axis=1)
    zeros_d = jnp.zeros((_RPT, _D), jnp.float32)
    ones_d = jnp.ones((_ECH, _D), jnp.float32)
    batch2d = batch.reshape(_N, 1)

    deg_parts = _degk(pk, ones_d, zeros_d)
    deg, dis, hd = _tc_prep(deg_parts, x)

    h = x
    p = params
    for l in (1, 2, 3):
        s_parts = _spmm(h, pk, zeros_d)
        z1p = _spmm(hd, pk, zeros_d)
        t1, q2 = _tc_mid(z1p, dis)
        z2p = _spmm(q2, pk, zeros_d)
        t2, q3 = _tc_mid(z2p, dis)
        z3p = _spmm(q3, pk, zeros_d)
        u1, st1 = _tc_tag(
            h, t1, t2, z3p, dis, p[f"tag{l}_W"], p[f"tag{l}_b"].reshape(1, _D),
        )
        u2, u3, st23 = _tc_lesage(
            h, s_parts, deg,
            p[f"le{l}_W1"], p[f"le{l}_b1"].reshape(1, _D),
            p[f"le{l}_W2"], p[f"le{l}_W3"], p[f"le{l}_b3"].reshape(1, _D),
            p[f"sage{l}_Wl"], p[f"sage{l}_bl"].reshape(1, _D), p[f"sage{l}_Wr"],
        )
        if l < 3:
            h, hd = _tc_apply(
                h, u1, u2, u3, st1, st23, dis,
                p[f"skip{l}_Wci"], p[f"skip{l}_bci"].reshape(1, _D),
                p[f"skip{l}_Wco"], p[f"skip{l}_bco"].reshape(1, _D),
                p[f"bn{l}1_g"].reshape(1, _D), p[f"bn{l}1_b"].reshape(1, _D),
                p[f"bn{l}2_g"].reshape(1, _D), p[f"bn{l}2_b"].reshape(1, _D),
                p[f"bn{l}3_g"].reshape(1, _D), p[f"bn{l}3_b"].reshape(1, _D),
            )
        else:
            fc1_W = params["fc1_W"]
            out = _tc_apply_out(
                h, u1, u2, u3, st1, st23,
                p[f"skip{l}_Wci"], p[f"skip{l}_bci"].reshape(1, _D),
                p[f"skip{l}_Wco"], p[f"skip{l}_bco"].reshape(1, _D),
                p[f"bn{l}1_g"].reshape(1, _D), p[f"bn{l}1_b"].reshape(1, _D),
                p[f"bn{l}2_g"].reshape(1, _D), p[f"bn{l}2_b"].reshape(1, _D),
                p[f"bn{l}3_g"].reshape(1, _D), p[f"bn{l}3_b"].reshape(1, _D),
                batch2d, eFeature,
                fc1_W[:_D], fc1_W[_D:],
                params["fc1_b"].reshape(1, _D),
                params["fc3_W"], params["fc3_b"].reshape(1, 1),
            )
    return out


# speedup vs baseline: 2.2096x; 1.0430x over previous
import functools

import jax
import jax.numpy as jnp
from jax import lax
from jax.experimental import pallas as pl
from jax.experimental.pallas import tpu as pltpu
from jax.experimental.pallas import tpu_sc as plsc

_N = 10000
_E = 320000
_D = 128
_G = 64
_K = 3

_NC = 2
_NS = 16
_NW = _NC * _NS
_EPW = _E // _NW
_CH = 128
_NFULL = _EPW // _CH
_TAIL = _EPW - _NFULL * _CH
_RPT = 632
_NP = _N


_ECH = 128
_NCHUNK = _E // _ECH
_CPW = _NCHUNK // _NW
_NEXTRA = _NCHUNK - _CPW * _NW
_NRB = 3
_NIB = 3
_UNROLL = 3


def _make_spmm(width):
    mesh = plsc.VectorSubcoreMesh(core_axis_name="c", subcore_axis_name="s")

    NI = 4
    NR = 3
    UN = 12

    scratch = [pltpu.VMEM_SHARED((_NP, width), jnp.float32)]
    scratch += [pltpu.VMEM((2, _ECH), jnp.int32) for _ in range(NI)]
    scratch += [pltpu.VMEM((_ECH, width), jnp.float32) for _ in range(NR)]
    scratch += [pltpu.SemaphoreType.DMA for _ in range(NI + 2 * NR)]

    @functools.partial(
        pl.kernel,
        out_type=jax.ShapeDtypeStruct((_NC * _NP, width), jnp.float32),
        mesh=mesh,
        scratch_types=scratch,
    )
    def spmm(h_hbm, pk_hbm, zeros_hbm, out_hbm, acc, *scr):
        idx = scr[0:NI]
        rows = scr[NI:NI + NR]
        isem = scr[NI + NR:2 * NI + NR]
        gsem = scr[2 * NI + NR:2 * NI + 2 * NR]
        ssem = scr[2 * NI + 2 * NR:2 * NI + 3 * NR]
        c = lax.axis_index("c")
        s = lax.axis_index("s")
        wid = s * _NC + c
        base = wid * _CPW

        def istart(ch, ib):
            pltpu.async_copy(pk_hbm.at[ch], idx[ib], isem[ib])

        def iwait(ch, ib):
            pltpu.make_async_copy(pk_hbm.at[ch], idx[ib], isem[ib]).wait()

        def gstart(rb, ib):
            pltpu.async_copy(h_hbm.at[idx[ib].at[0]], rows[rb], gsem[rb])

        def gwait(rb, ib):
            pltpu.make_async_copy(h_hbm.at[idx[ib].at[0]], rows[rb],
                                  gsem[rb]).wait()

        def sstart(rb, ib):
            pltpu.async_copy(rows[rb], acc.at[idx[ib].at[1]], ssem[rb],
                             add=True)

        def swait(rb, ib):
            pltpu.make_async_copy(rows[rb], acc.at[idx[ib].at[1]],
                                  ssem[rb]).wait()

        def step(j, k, last):
            rbg = (k + 1) % NR
            ibg = (k + 1) % NI

            def _swait_jm2():
                swait(rbg, (k + 2) % NI)

            def _gnext():
                iwait(base + j + 1, ibg)
                gstart(rbg, ibg)

            def _inext():
                istart(base + j + 2, (k + 2) % NI)

            if last is None:
                pl.when(j >= 2)(_swait_jm2)
                pl.when(j + 1 < _CPW)(_gnext)
                pl.when(j + 2 < _CPW)(_inext)
            else:
                if j >= 2:
                    _swait_jm2()
                if j + 1 < _CPW:
                    _gnext()
                if j + 2 < _CPW:
                    _inext()
            gwait(k % NR, k % NI)
            sstart(k % NR, k % NI)

        @pl.when(s < 2)
        def _():
            pltpu.sync_copy(zeros_hbm, acc.at[pl.ds(s * 632, 632)])

        @pl.when(s >= 2)
        def _():
            pltpu.sync_copy(zeros_hbm.at[pl.ds(0, 624)],
                            acc.at[pl.ds(624 * s + 16, 624)])

        istart(base + 0, 0)
        istart(base + 1, 1)
        iwait(base + 0, 0)
        gstart(0, 0)
        plsc.subcore_barrier()

        def outer(i, carry):
            jb = i * UN
            for k in range(UN):
                step(jb + k, k, None)
            return carry

        nmain = (_CPW // UN) * UN
        lax.fori_loop(0, _CPW // UN, outer, 0)
        for j in range(nmain, _CPW):
            step(j, j % UN, True)
        swait((_CPW - 2) % NR, (_CPW - 2) % NI)
        swait((_CPW - 1) % NR, (_CPW - 1) % NI)

        @pl.when(wid < _NEXTRA)
        def _():
            ch = _NW * _CPW + wid
            istart(ch, 0)
            iwait(ch, 0)
            gstart(0, 0)
            gwait(0, 0)
            sstart(0, 0)
            swait(0, 0)

        plsc.subcore_barrier()

        @pl.when(s < 2)
        def _():
            pltpu.sync_copy(acc.at[pl.ds(s * 632, 632)],
                            out_hbm.at[pl.ds(c * _NP + s * 632, 632)])

        @pl.when(s >= 2)
        def _():
            pltpu.sync_copy(acc.at[pl.ds(624 * s + 16, 624)],
                            out_hbm.at[pl.ds(c * _NP + 624 * s + 16, 624)])

    return spmm


def _make_deg():
    width = _D
    mesh = plsc.VectorSubcoreMesh(core_axis_name="c", subcore_axis_name="s")

    scratch = [pltpu.VMEM_SHARED((_NP, width), jnp.float32)]
    scratch += [pltpu.VMEM((2, _ECH), jnp.int32) for _ in range(_NIB)]
    scratch += [pltpu.VMEM((_ECH, width), jnp.float32)]
    scratch += [pltpu.SemaphoreType.DMA for _ in range(2 * _NIB)]

    @functools.partial(
        pl.kernel,
        out_type=jax.ShapeDtypeStruct((_NC * _NP, width), jnp.float32),
        mesh=mesh,
        scratch_types=scratch,
    )
    def degk(pk_hbm, ones_hbm, zeros_hbm, out_hbm, acc, *scr):
        idx = scr[0:_NIB]
        rows = scr[_NIB]
        isem = scr[_NIB + 1:2 * _NIB + 1]
        ssem = scr[2 * _NIB + 1:3 * _NIB + 1]
        c = lax.axis_index("c")
        s = lax.axis_index("s")
        wid = s * _NC + c
        base = wid * _CPW

        def istart(ch, b):
            pltpu.async_copy(pk_hbm.at[ch], idx[b], isem[b])

        def iwait(ch, b):
            pltpu.make_async_copy(pk_hbm.at[ch], idx[b], isem[b]).wait()

        def sstart(b):
            pltpu.async_copy(rows, acc.at[idx[b].at[1]], ssem[b], add=True)

        def swait(b):
            pltpu.make_async_copy(rows, acc.at[idx[b].at[1]],
                                  ssem[b]).wait()

        @pl.when(s < 2)
        def _():
            pltpu.sync_copy(zeros_hbm, acc.at[pl.ds(s * 632, 632)])

        @pl.when(s >= 2)
        def _():
            pltpu.sync_copy(zeros_hbm.at[pl.ds(0, 624)],
                            acc.at[pl.ds(624 * s + 16, 624)])

        pltpu.sync_copy(ones_hbm, rows)
        plsc.subcore_barrier()

        istart(base + 0, 0)
        istart(base + 1, 1)
        iwait(base + 0, 0)

        def outer(i, carry):
            jb = i * _UNROLL
            for k in range(_UNROLL):
                j = jb + k
                b = k % _NIB
                b1 = (k + 1) % _NIB
                b2 = (k + 2) % _NIB

                @pl.when(j + 1 < _CPW)
                def _():
                    iwait(base + j + 1, b1)

                @pl.when(j >= 1)
                def _():
                    swait(b2)

                @pl.when(j + 2 < _CPW)
                def _():
                    istart(base + j + 2, b2)

                sstart(b)
            return carry

        lax.fori_loop(0, _CPW // _UNROLL, outer, 0)
        swait((_CPW - 1) % _NIB)

        @pl.when(wid < _NEXTRA)
        def _():
            ch = _NW * _CPW + wid
            istart(ch, 0)
            iwait(ch, 0)
            sstart(0)
            swait(0)

        plsc.subcore_barrier()

        @pl.when(s < 2)
        def _():
            pltpu.sync_copy(acc.at[pl.ds(s * 632, 632)],
                            out_hbm.at[pl.ds(c * _NP + s * 632, 632)])

        @pl.when(s >= 2)
        def _():
            pltpu.sync_copy(acc.at[pl.ds(624 * s + 16, 624)],
                            out_hbm.at[pl.ds(c * _NP + 624 * s + 16, 624)])

    return degk


_spmm = _make_spmm(_D)
_degk = _make_deg()


def _prep_body(dp_ref, x_ref, deg_ref, dis_ref, hd_ref):
    deg = dp_ref[0:_N, 0:1] + dp_ref[_NP:_NP + _N, 0:1]
    deg_ref[...] = deg
    dis = jnp.where(deg > 0, 1.0 / jnp.sqrt(jnp.maximum(deg, 1e-12)), 0.0)
    dis_ref[...] = dis
    hd_ref[...] = x_ref[...] * dis


_tc_prep = pl.pallas_call(
    _prep_body,
    out_shape=(
        jax.ShapeDtypeStruct((_N, 1), jnp.float32),
        jax.ShapeDtypeStruct((_N, 1), jnp.float32),
        jax.ShapeDtypeStruct((_N, _D), jnp.float32),
    ),
)


def _mid_body(zp_ref, dis_ref, t_ref, q_ref):
    z = zp_ref[0:_N, :] + zp_ref[_NP:_NP + _N, :]
    dis = dis_ref[...]
    t = z * dis
    t_ref[...] = t
    q_ref[...] = t * dis


_tc_mid = pl.pallas_call(
    _mid_body,
    out_shape=(
        jax.ShapeDtypeStruct((_N, _D), jnp.float32),
        jax.ShapeDtypeStruct((_N, _D), jnp.float32),
    ),
)


def _mm(a, b):
    return jnp.dot(a, b, preferred_element_type=jnp.float32)


def _bn(u, g, b):
    m = jnp.mean(u, axis=0, keepdims=True)
    v = jnp.mean((u - m) * (u - m), axis=0, keepdims=True)
    return (u - m) / jnp.sqrt(v + 1e-5) * g + b


def _stats(u):
    m = jnp.mean(u, axis=0, keepdims=True)
    v = jnp.mean((u - m) * (u - m), axis=0, keepdims=True)
    return m, v


def _tag_body(h_ref, t1_ref, t2_ref, z3p_ref, dis_ref, tagW_ref, tagb_ref,
              u1_ref, st_ref):
    t3 = (z3p_ref[0:_N, :] + z3p_ref[_NP:_NP + _N, :]) * dis_ref[...]
    u1 = (_mm(h_ref[...], tagW_ref[0]) + _mm(t1_ref[...], tagW_ref[1])
          + _mm(t2_ref[...], tagW_ref[2]) + _mm(t3, tagW_ref[3])
          + tagb_ref[...])
    u1_ref[...] = u1
    m1, v1 = _stats(u1)
    st_ref[...] = jnp.concatenate([m1, v1], axis=0)


_tc_tag = pl.pallas_call(
    _tag_body,
    out_shape=(
        jax.ShapeDtypeStruct((_N, _D), jnp.float32),
        jax.ShapeDtypeStruct((2, _D), jnp.float32),
    ),
)


def _lesage_body(h_ref, sp_ref, deg_ref,
                 leW1_ref, leb1_ref, leW2_ref, leW3_ref, leb3_ref,
                 sgWl_ref, sgbl_ref, sgWr_ref,
                 u2_ref, u3_ref, st_ref):
    h = h_ref[...]
    s = sp_ref[0:_N, :] + sp_ref[_NP:_NP + _N, :]
    deg = deg_ref[...]
    u2 = (deg * (_mm(h, leW1_ref[...]) + leb1_ref[...])
          - _mm(s, leW2_ref[...]) + _mm(h, leW3_ref[...]) + leb3_ref[...])
    u3 = (_mm(s / jnp.maximum(deg, 1.0), sgWl_ref[...]) + sgbl_ref[...]
          + _mm(h, sgWr_ref[...]))
    u2_ref[...] = u2
    u3_ref[...] = u3
    m2, v2 = _stats(u2)
    m3, v3 = _stats(u3)
    st_ref[...] = jnp.concatenate([m2, v2, m3, v3], axis=0)


_tc_lesage = pl.pallas_call(
    _lesage_body,
    out_shape=(
        jax.ShapeDtypeStruct((_N, _D), jnp.float32),
        jax.ShapeDtypeStruct((_N, _D), jnp.float32),
        jax.ShapeDtypeStruct((4, _D), jnp.float32),
    ),
)


def _apply_body(h_ref, u1_ref, u2_ref, u3_ref, st1_ref, st23_ref, dis_ref,
                skWci_ref, skbci_ref, skWco_ref, skbco_ref,
                bn1g_ref, bn1b_ref, bn2g_ref, bn2b_ref, bn3g_ref, bn3b_ref,
                hn_ref, hdn_ref):
    h = h_ref[...]
    dis = dis_ref[...]

    def norm(u, m, v, g, b):
        return jnp.maximum((u - m) / jnp.sqrt(v + 1e-5) * g + b, 0.0)

    o = (norm(u1_ref[...], st1_ref[0:1, :], st1_ref[1:2, :],
              bn1g_ref[...], bn1b_ref[...])
         + norm(u2_ref[...], st23_ref[0:1, :], st23_ref[1:2, :],
                bn2g_ref[...], bn2b_ref[...])
         + norm(u3_ref[...], st23_ref[2:3, :], st23_ref[3:4, :],
                bn3g_ref[...], bn3b_ref[...]))
    zl = (_mm(h, skWci_ref[...]) + skbci_ref[...]
          + _mm(o, skWco_ref[...]) + skbco_ref[...])
    z = 1.0 / (1.0 + jnp.exp(-zl))
    hn = z * o + (1.0 - z) * h
    hn_ref[...] = hn
    hdn_ref[...] = hn * dis


_tc_apply = pl.pallas_call(
    _apply_body,
    out_shape=(
        jax.ShapeDtypeStruct((_N, _D), jnp.float32),
        jax.ShapeDtypeStruct((_N, _D), jnp.float32),
    ),
)


def _apply_out_body(h_ref, u1_ref, u2_ref, u3_ref, st1_ref, st23_ref,
                    skWci_ref, skbci_ref, skWco_ref, skbco_ref,
                    bn1g_ref, bn1b_ref, bn2g_ref, bn2b_ref,
                    bn3g_ref, bn3b_ref,
                    batch_ref, eF_ref, w1h_ref, w1e_ref, b1_ref,
                    w3_ref, b3_ref, out_ref):
    h = h_ref[...]

    def norm(u, m, v, g, b):
        return jnp.maximum((u - m) / jnp.sqrt(v + 1e-5) * g + b, 0.0)

    o = (norm(u1_ref[...], st1_ref[0:1, :], st1_ref[1:2, :],
              bn1g_ref[...], bn1b_ref[...])
         + norm(u2_ref[...], st23_ref[0:1, :], st23_ref[1:2, :],
                bn2g_ref[...], bn2b_ref[...])
         + norm(u3_ref[...], st23_ref[2:3, :], st23_ref[3:4, :],
                bn3g_ref[...], bn3b_ref[...]))
    zl = (_mm(h, skWci_ref[...]) + skbci_ref[...]
          + _mm(o, skWco_ref[...]) + skbco_ref[...])
    z = 1.0 / (1.0 + jnp.exp(-zl))
    hn = z * o + (1.0 - z) * h

    gids = lax.broadcasted_iota(jnp.int32, (1, _G), 1)
    onehot = (batch_ref[...] == gids).astype(jnp.float32)
    sums = lax.dot_general(onehot, hn, (((0,), (0,)), ((), ())),
                           preferred_element_type=jnp.float32)
    ones_col = jnp.ones((_N, 1), jnp.float32)
    cnts = lax.dot_general(onehot, ones_col, (((0,), (0,)), ((), ())),
                           preferred_element_type=jnp.float32)
    hg = sums / jnp.maximum(cnts, 1.0)
    r = _mm(hg, w1h_ref[...]) + _mm(eF_ref[...], w1e_ref[...]) + b1_ref[...]
    r = jnp.maximum(r, 0.0)
    out_ref[...] = _mm(r, w3_ref[...]) + b3_ref[...]


_tc_apply_out = pl.pallas_call(
    _apply_out_body,
    out_shape=jax.ShapeDtypeStruct((_G, 1), jnp.float32),
)


def _readout_body(h_ref, batch_ref, eF_ref, w1h_ref, w1e_ref, b1_ref,
                  w3_ref, b3_ref, out_ref):
    h = h_ref[...]
    gids = lax.broadcasted_iota(jnp.int32, (1, _G), 1)
    onehot = (batch_ref[...] == gids).astype(jnp.float32)
    sums = lax.dot_general(onehot, h, (((0,), (0,)), ((), ())),
                           preferred_element_type=jnp.float32)
    ones_col = jnp.ones((_N, 1), jnp.float32)
    cnts = lax.dot_general(onehot, ones_col, (((0,), (0,)), ((), ())),
                           preferred_element_type=jnp.float32)
    hg = sums / jnp.maximum(cnts, 1.0)
    r = _mm(hg, w1h_ref[...]) + _mm(eF_ref[...], w1e_ref[...]) + b1_ref[...]
    r = jnp.maximum(r, 0.0)
    out_ref[...] = _mm(r, w3_ref[...]) + b3_ref[...]


_tc_readout = pl.pallas_call(
    _readout_body,
    out_shape=jax.ShapeDtypeStruct((_G, 1), jnp.float32),
)


def kernel(x, edge_index, batch, eFeature, params):
    src = edge_index[0]
    dst = edge_index[1]
    pk = jnp.stack([src.reshape(_NCHUNK, _ECH), dst.reshape(_NCHUNK, _ECH)],
                   axis=1)
    zeros_d = jnp.zeros((_RPT, _D), jnp.float32)
    ones_d = jnp.ones((_ECH, _D), jnp.float32)
    batch2d = batch.reshape(_N, 1)

    deg_parts = _degk(pk, ones_d, zeros_d)
    deg, dis, hd = _tc_prep(deg_parts, x)

    h = x
    p = params
    for l in (1, 2, 3):
        s_parts = _spmm(h, pk, zeros_d)
        z1p = _spmm(hd, pk, zeros_d)
        t1, q2 = _tc_mid(z1p, dis)
        z2p = _spmm(q2, pk, zeros_d)
        t2, q3 = _tc_mid(z2p, dis)
        z3p = _spmm(q3, pk, zeros_d)
        u1, st1 = _tc_tag(
            h, t1, t2, z3p, dis, p[f"tag{l}_W"], p[f"tag{l}_b"].reshape(1, _D),
        )
        u2, u3, st23 = _tc_lesage(
            h, s_parts, deg,
            p[f"le{l}_W1"], p[f"le{l}_b1"].reshape(1, _D),
            p[f"le{l}_W2"], p[f"le{l}_W3"], p[f"le{l}_b3"].reshape(1, _D),
            p[f"sage{l}_Wl"], p[f"sage{l}_bl"].reshape(1, _D), p[f"sage{l}_Wr"],
        )
        if l < 3:
            h, hd = _tc_apply(
                h, u1, u2, u3, st1, st23, dis,
                p[f"skip{l}_Wci"], p[f"skip{l}_bci"].reshape(1, _D),
                p[f"skip{l}_Wco"], p[f"skip{l}_bco"].reshape(1, _D),
                p[f"bn{l}1_g"].reshape(1, _D), p[f"bn{l}1_b"].reshape(1, _D),
                p[f"bn{l}2_g"].reshape(1, _D), p[f"bn{l}2_b"].reshape(1, _D),
                p[f"bn{l}3_g"].reshape(1, _D), p[f"bn{l}3_b"].reshape(1, _D),
            )
        else:
            fc1_W = params["fc1_W"]
            out = _tc_apply_out(
                h, u1, u2, u3, st1, st23,
                p[f"skip{l}_Wci"], p[f"skip{l}_bci"].reshape(1, _D),
                p[f"skip{l}_Wco"], p[f"skip{l}_bco"].reshape(1, _D),
                p[f"bn{l}1_g"].reshape(1, _D), p[f"bn{l}1_b"].reshape(1, _D),
                p[f"bn{l}2_g"].reshape(1, _D), p[f"bn{l}2_b"].reshape(1, _D),
                p[f"bn{l}3_g"].reshape(1, _D), p[f"bn{l}3_b"].reshape(1, _D),
                batch2d, eFeature,
                fc1_W[:_D], fc1_W[_D:],
                params["fc1_b"].reshape(1, _D),
                params["fc3_W"], params["fc3_b"].reshape(1, 1),
            )
    return out


# speedup vs baseline: 2.2101x; 1.0002x over previous
import functools

import jax
import jax.numpy as jnp
from jax import lax
from jax.experimental import pallas as pl
from jax.experimental.pallas import tpu as pltpu
from jax.experimental.pallas import tpu_sc as plsc

_N = 10000
_E = 320000
_D = 128
_G = 64
_K = 3

_NC = 2
_NS = 16
_NW = _NC * _NS
_EPW = _E // _NW
_CH = 128
_NFULL = _EPW // _CH
_TAIL = _EPW - _NFULL * _CH
_RPT = 632
_NP = _N


_ECH = 128
_NCHUNK = _E // _ECH
_CPW = _NCHUNK // _NW
_NEXTRA = _NCHUNK - _CPW * _NW
_NRB = 3
_NIB = 3
_UNROLL = 3


def _make_spmm(width):
    mesh = plsc.VectorSubcoreMesh(core_axis_name="c", subcore_axis_name="s")

    NI = 4
    NR = 3
    UN = 12

    scratch = [pltpu.VMEM_SHARED((_NP, width), jnp.float32)]
    scratch += [pltpu.VMEM((2, _ECH), jnp.int32) for _ in range(NI)]
    scratch += [pltpu.VMEM((_ECH, width), jnp.float32) for _ in range(NR)]
    scratch += [pltpu.SemaphoreType.DMA for _ in range(NI + 2 * NR)]

    @functools.partial(
        pl.kernel,
        out_type=jax.ShapeDtypeStruct((_NC * _NP, width), jnp.float32),
        mesh=mesh,
        scratch_types=scratch,
    )
    def spmm(h_hbm, pk_hbm, zeros_hbm, out_hbm, acc, *scr):
        idx = scr[0:NI]
        rows = scr[NI:NI + NR]
        isem = scr[NI + NR:2 * NI + NR]
        gsem = scr[2 * NI + NR:2 * NI + 2 * NR]
        ssem = scr[2 * NI + 2 * NR:2 * NI + 3 * NR]
        c = lax.axis_index("c")
        s = lax.axis_index("s")
        wid = s * _NC + c
        base = wid * _CPW

        def istart(ch, ib):
            pltpu.async_copy(pk_hbm.at[ch], idx[ib], isem[ib])

        def iwait(ch, ib):
            pltpu.make_async_copy(pk_hbm.at[ch], idx[ib], isem[ib]).wait()

        def gstart(rb, ib):
            pltpu.async_copy(h_hbm.at[idx[ib].at[0]], rows[rb], gsem[rb])

        def gwait(rb, ib):
            pltpu.make_async_copy(h_hbm.at[idx[ib].at[0]], rows[rb],
                                  gsem[rb]).wait()

        def sstart(rb, ib):
            pltpu.async_copy(rows[rb], acc.at[idx[ib].at[1]], ssem[rb],
                             add=True)

        def swait(rb, ib):
            pltpu.make_async_copy(rows[rb], acc.at[idx[ib].at[1]],
                                  ssem[rb]).wait()

        def step(j, k, last):
            rbg = (k + 1) % NR
            ibg = (k + 1) % NI

            def _swait_jm2():
                swait(rbg, (k + 2) % NI)

            def _gnext():
                iwait(base + j + 1, ibg)
                gstart(rbg, ibg)

            def _inext():
                istart(base + j + 2, (k + 2) % NI)

            if last is None:
                pl.when(j >= 2)(_swait_jm2)
                pl.when(j + 1 < _CPW)(_gnext)
                pl.when(j + 2 < _CPW)(_inext)
            else:
                if j >= 2:
                    _swait_jm2()
                if j + 1 < _CPW:
                    _gnext()
                if j + 2 < _CPW:
                    _inext()
            gwait(k % NR, k % NI)
            sstart(k % NR, k % NI)

        @pl.when(s < 2)
        def _():
            pltpu.sync_copy(zeros_hbm, acc.at[pl.ds(s * 632, 632)])

        @pl.when(s >= 2)
        def _():
            pltpu.sync_copy(zeros_hbm.at[pl.ds(0, 624)],
                            acc.at[pl.ds(624 * s + 16, 624)])

        istart(base + 0, 0)
        istart(base + 1, 1)
        iwait(base + 0, 0)
        gstart(0, 0)
        plsc.subcore_barrier()

        def outer(i, carry):
            jb = i * UN
            for k in range(UN):
                step(jb + k, k, None)
            return carry

        nmain = (_CPW // UN) * UN
        lax.fori_loop(0, _CPW // UN, outer, 0)
        for j in range(nmain, _CPW):
            step(j, j % UN, True)
        swait((_CPW - 2) % NR, (_CPW - 2) % NI)
        swait((_CPW - 1) % NR, (_CPW - 1) % NI)

        @pl.when(wid < _NEXTRA)
        def _():
            ch = _NW * _CPW + wid
            istart(ch, 0)
            iwait(ch, 0)
            gstart(0, 0)
            gwait(0, 0)
            sstart(0, 0)
            swait(0, 0)

        plsc.subcore_barrier()

        @pl.when(s < 2)
        def _():
            pltpu.sync_copy(acc.at[pl.ds(s * 632, 632)],
                            out_hbm.at[pl.ds(c * _NP + s * 632, 632)])

        @pl.when(s >= 2)
        def _():
            pltpu.sync_copy(acc.at[pl.ds(624 * s + 16, 624)],
                            out_hbm.at[pl.ds(c * _NP + 624 * s + 16, 624)])

    return spmm


def _make_deg():
    width = _D
    mesh = plsc.VectorSubcoreMesh(core_axis_name="c", subcore_axis_name="s")

    scratch = [pltpu.VMEM_SHARED((_NP, width), jnp.float32)]
    scratch += [pltpu.VMEM((2, _ECH), jnp.int32) for _ in range(_NIB)]
    scratch += [pltpu.VMEM((_ECH, width), jnp.float32)]
    scratch += [pltpu.SemaphoreType.DMA for _ in range(2 * _NIB)]

    @functools.partial(
        pl.kernel,
        out_type=jax.ShapeDtypeStruct((_NC * _NP, width), jnp.float32),
        mesh=mesh,
        scratch_types=scratch,
    )
    def degk(pk_hbm, ones_hbm, zeros_hbm, out_hbm, acc, *scr):
        idx = scr[0:_NIB]
        rows = scr[_NIB]
        isem = scr[_NIB + 1:2 * _NIB + 1]
        ssem = scr[2 * _NIB + 1:3 * _NIB + 1]
        c = lax.axis_index("c")
        s = lax.axis_index("s")
        wid = s * _NC + c
        base = wid * _CPW

        def istart(ch, b):
            pltpu.async_copy(pk_hbm.at[ch], idx[b], isem[b])

        def iwait(ch, b):
            pltpu.make_async_copy(pk_hbm.at[ch], idx[b], isem[b]).wait()

        def sstart(b):
            pltpu.async_copy(rows, acc.at[idx[b].at[1]], ssem[b], add=True)

        def swait(b):
            pltpu.make_async_copy(rows, acc.at[idx[b].at[1]],
                                  ssem[b]).wait()

        @pl.when(s < 2)
        def _():
            pltpu.sync_copy(zeros_hbm, acc.at[pl.ds(s * 632, 632)])

        @pl.when(s >= 2)
        def _():
            pltpu.sync_copy(zeros_hbm.at[pl.ds(0, 624)],
                            acc.at[pl.ds(624 * s + 16, 624)])

        pltpu.sync_copy(ones_hbm, rows)
        plsc.subcore_barrier()

        istart(base + 0, 0)
        istart(base + 1, 1)
        iwait(base + 0, 0)

        def outer(i, carry):
            jb = i * _UNROLL
            for k in range(_UNROLL):
                j = jb + k
                b = k % _NIB
                b1 = (k + 1) % _NIB
                b2 = (k + 2) % _NIB

                @pl.when(j + 1 < _CPW)
                def _():
                    iwait(base + j + 1, b1)

                @pl.when(j >= 1)
                def _():
                    swait(b2)

                @pl.when(j + 2 < _CPW)
                def _():
                    istart(base + j + 2, b2)

                sstart(b)
            return carry

        lax.fori_loop(0, _CPW // _UNROLL, outer, 0)
        swait((_CPW - 1) % _NIB)

        @pl.when(wid < _NEXTRA)
        def _():
            ch = _NW * _CPW + wid
            istart(ch, 0)
            iwait(ch, 0)
            sstart(0)
            swait(0)

        plsc.subcore_barrier()

        @pl.when(s < 2)
        def _():
            pltpu.sync_copy(acc.at[pl.ds(s * 632, 632)],
                            out_hbm.at[pl.ds(c * _NP + s * 632, 632)])

        @pl.when(s >= 2)
        def _():
            pltpu.sync_copy(acc.at[pl.ds(624 * s + 16, 624)],
                            out_hbm.at[pl.ds(c * _NP + 624 * s + 16, 624)])

    return degk


_spmm = _make_spmm(_D)
_degk = _make_deg()


def _prep_body(dp_ref, x_ref, deg_ref, dis_ref, hd_ref):
    deg = dp_ref[0:_N, 0:1] + dp_ref[_NP:_NP + _N, 0:1]
    deg_ref[...] = deg
    dis = jnp.where(deg > 0, 1.0 / jnp.sqrt(jnp.maximum(deg, 1e-12)), 0.0)
    dis_ref[...] = dis
    hd_ref[...] = x_ref[...] * dis


_tc_prep = pl.pallas_call(
    _prep_body,
    out_shape=(
        jax.ShapeDtypeStruct((_N, 1), jnp.float32),
        jax.ShapeDtypeStruct((_N, 1), jnp.float32),
        jax.ShapeDtypeStruct((_N, _D), jnp.float32),
    ),
)


def _mid_body(zp_ref, dis_ref, t_ref, q_ref):
    z = zp_ref[0:_N, :] + zp_ref[_NP:_NP + _N, :]
    dis = dis_ref[...]
    t = z * dis
    t_ref[...] = t
    q_ref[...] = t * dis


_tc_mid = pl.pallas_call(
    _mid_body,
    out_shape=(
        jax.ShapeDtypeStruct((_N, _D), jnp.float32),
        jax.ShapeDtypeStruct((_N, _D), jnp.float32),
    ),
)


def _mm(a, b):
    return jnp.dot(a, b, preferred_element_type=jnp.float32)


def _stats(u):
    m = jnp.mean(u, axis=0, keepdims=True)
    v = jnp.mean((u - m) * (u - m), axis=0, keepdims=True)
    return m, v


def _tag_body(h_ref, t1_ref, t2_ref, z3p_ref, dis_ref, tagW_ref, tagb_ref,
              u1_ref, st_ref):
    t3 = (z3p_ref[0:_N, :] + z3p_ref[_NP:_NP + _N, :]) * dis_ref[...]
    u1 = (_mm(h_ref[...], tagW_ref[0]) + _mm(t1_ref[...], tagW_ref[1])
          + _mm(t2_ref[...], tagW_ref[2]) + _mm(t3, tagW_ref[3])
          + tagb_ref[...])
    u1_ref[...] = u1
    m1, v1 = _stats(u1)
    st_ref[...] = jnp.concatenate([m1, v1], axis=0)


_tc_tag = pl.pallas_call(
    _tag_body,
    out_shape=(
        jax.ShapeDtypeStruct((_N, _D), jnp.float32),
        jax.ShapeDtypeStruct((2, _D), jnp.float32),
    ),
)


def _lesage_body(h_ref, sp_ref, deg_ref,
                 leW1_ref, leb1_ref, leW2_ref, leW3_ref, leb3_ref,
                 sgWl_ref, sgbl_ref, sgWr_ref,
                 u2_ref, u3_ref, st_ref):
    h = h_ref[...]
    s = sp_ref[0:_N, :] + sp_ref[_NP:_NP + _N, :]
    deg = deg_ref[...]
    u2 = (deg * (_mm(h, leW1_ref[...]) + leb1_ref[...])
          - _mm(s, leW2_ref[...]) + _mm(h, leW3_ref[...]) + leb3_ref[...])
    u3 = (_mm(s / jnp.maximum(deg, 1.0), sgWl_ref[...]) + sgbl_ref[...]
          + _mm(h, sgWr_ref[...]))
    u2_ref[...] = u2
    u3_ref[...] = u3
    m2, v2 = _stats(u2)
    m3, v3 = _stats(u3)
    st_ref[...] = jnp.concatenate([m2, v2, m3, v3], axis=0)


_tc_lesage = pl.pallas_call(
    _lesage_body,
    out_shape=(
        jax.ShapeDtypeStruct((_N, _D), jnp.float32),
        jax.ShapeDtypeStruct((_N, _D), jnp.float32),
        jax.ShapeDtypeStruct((4, _D), jnp.float32),
    ),
)


def _apply_body(h_ref, u1_ref, u2_ref, u3_ref, st1_ref, st23_ref, dis_ref,
                skWci_ref, skbci_ref, skWco_ref, skbco_ref,
                bn1g_ref, bn1b_ref, bn2g_ref, bn2b_ref, bn3g_ref, bn3b_ref,
                hn_ref, hdn_ref):
    h = h_ref[...]
    dis = dis_ref[...]

    def norm(u, m, v, g, b):
        return jnp.maximum((u - m) / jnp.sqrt(v + 1e-5) * g + b, 0.0)

    o = (norm(u1_ref[...], st1_ref[0:1, :], st1_ref[1:2, :],
              bn1g_ref[...], bn1b_ref[...])
         + norm(u2_ref[...], st23_ref[0:1, :], st23_ref[1:2, :],
                bn2g_ref[...], bn2b_ref[...])
         + norm(u3_ref[...], st23_ref[2:3, :], st23_ref[3:4, :],
                bn3g_ref[...], bn3b_ref[...]))
    zl = (_mm(h, skWci_ref[...]) + skbci_ref[...]
          + _mm(o, skWco_ref[...]) + skbco_ref[...])
    z = 1.0 / (1.0 + jnp.exp(-zl))
    hn = z * o + (1.0 - z) * h
    hn_ref[...] = hn
    hdn_ref[...] = hn * dis


_tc_apply = pl.pallas_call(
    _apply_body,
    out_shape=(
        jax.ShapeDtypeStruct((_N, _D), jnp.float32),
        jax.ShapeDtypeStruct((_N, _D), jnp.float32),
    ),
)


def _apply_out_body(h_ref, u1_ref, u2_ref, u3_ref, st1_ref, st23_ref,
                    skWci_ref, skbci_ref, skWco_ref, skbco_ref,
                    bn1g_ref, bn1b_ref, bn2g_ref, bn2b_ref,
                    bn3g_ref, bn3b_ref,
                    batch_ref, eF_ref, w1h_ref, w1e_ref, b1_ref,
                    w3_ref, b3_ref, out_ref):
    h = h_ref[...]

    def norm(u, m, v, g, b):
        return jnp.maximum((u - m) / jnp.sqrt(v + 1e-5) * g + b, 0.0)

    o = (norm(u1_ref[...], st1_ref[0:1, :], st1_ref[1:2, :],
              bn1g_ref[...], bn1b_ref[...])
         + norm(u2_ref[...], st23_ref[0:1, :], st23_ref[1:2, :],
                bn2g_ref[...], bn2b_ref[...])
         + norm(u3_ref[...], st23_ref[2:3, :], st23_ref[3:4, :],
                bn3g_ref[...], bn3b_ref[...]))
    zl = (_mm(h, skWci_ref[...]) + skbci_ref[...]
          + _mm(o, skWco_ref[...]) + skbco_ref[...])
    z = 1.0 / (1.0 + jnp.exp(-zl))
    hn = z * o + (1.0 - z) * h

    gids = lax.broadcasted_iota(jnp.int32, (1, _G), 1)
    onehot = (batch_ref[...] == gids).astype(jnp.float32)
    sums = lax.dot_general(onehot, hn, (((0,), (0,)), ((), ())),
                           preferred_element_type=jnp.float32)
    ones_col = jnp.ones((_N, 1), jnp.float32)
    cnts = lax.dot_general(onehot, ones_col, (((0,), (0,)), ((), ())),
                           preferred_element_type=jnp.float32)
    hg = sums / jnp.maximum(cnts, 1.0)
    r = _mm(hg, w1h_ref[...]) + _mm(eF_ref[...], w1e_ref[...]) + b1_ref[...]
    r = jnp.maximum(r, 0.0)
    out_ref[...] = _mm(r, w3_ref[...]) + b3_ref[...]


_tc_apply_out = pl.pallas_call(
    _apply_out_body,
    out_shape=jax.ShapeDtypeStruct((_G, 1), jnp.float32),
)


def kernel(x, edge_index, batch, eFeature, params):
    src = edge_index[0]
    dst = edge_index[1]
    pk = jnp.stack([src.reshape(_NCHUNK, _ECH), dst.reshape(_NCHUNK, _ECH)],
                   axis=1)
    zeros_d = jnp.zeros((_RPT, _D), jnp.float32)
    ones_d = jnp.ones((_ECH, _D), jnp.float32)
    batch2d = batch.reshape(_N, 1)

    deg_parts = _degk(pk, ones_d, zeros_d)
    deg, dis, hd = _tc_prep(deg_parts, x)

    h = x
    p = params
    for l in (1, 2, 3):
        s_parts = _spmm(h, pk, zeros_d)
        z1p = _spmm(hd, pk, zeros_d)
        t1, q2 = _tc_mid(z1p, dis)
        z2p = _spmm(q2, pk, zeros_d)
        t2, q3 = _tc_mid(z2p, dis)
        z3p = _spmm(q3, pk, zeros_d)
        u1, st1 = _tc_tag(
            h, t1, t2, z3p, dis, p[f"tag{l}_W"], p[f"tag{l}_b"].reshape(1, _D),
        )
        u2, u3, st23 = _tc_lesage(
            h, s_parts, deg,
            p[f"le{l}_W1"], p[f"le{l}_b1"].reshape(1, _D),
            p[f"le{l}_W2"], p[f"le{l}_W3"], p[f"le{l}_b3"].reshape(1, _D),
            p[f"sage{l}_Wl"], p[f"sage{l}_bl"].reshape(1, _D), p[f"sage{l}_Wr"],
        )
        if l < 3:
            h, hd = _tc_apply(
                h, u1, u2, u3, st1, st23, dis,
                p[f"skip{l}_Wci"], p[f"skip{l}_bci"].reshape(1, _D),
                p[f"skip{l}_Wco"], p[f"skip{l}_bco"].reshape(1, _D),
                p[f"bn{l}1_g"].reshape(1, _D), p[f"bn{l}1_b"].reshape(1, _D),
                p[f"bn{l}2_g"].reshape(1, _D), p[f"bn{l}2_b"].reshape(1, _D),
                p[f"bn{l}3_g"].reshape(1, _D), p[f"bn{l}3_b"].reshape(1, _D),
            )
        else:
            fc1_W = params["fc1_W"]
            out = _tc_apply_out(
                h, u1, u2, u3, st1, st23,
                p[f"skip{l}_Wci"], p[f"skip{l}_bci"].reshape(1, _D),
                p[f"skip{l}_Wco"], p[f"skip{l}_bco"].reshape(1, _D),
                p[f"bn{l}1_g"].reshape(1, _D), p[f"bn{l}1_b"].reshape(1, _D),
                p[f"bn{l}2_g"].reshape(1, _D), p[f"bn{l}2_b"].reshape(1, _D),
                p[f"bn{l}3_g"].reshape(1, _D), p[f"bn{l}3_b"].reshape(1, _D),
                batch2d, eFeature,
                fc1_W[:_D], fc1_W[_D:],
                params["fc1_b"].reshape(1, _D),
                params["fc3_W"], params["fc3_b"].reshape(1, 1),
            )
    return out
